# Initial kernel scaffold; baseline (speedup 1.0000x reference)
import jax
import jax.numpy as jnp
from jax.experimental import pallas as pl


def kernel(x, edge_index, W1l, W1r, b1, W2l, W2r, b2):
    raise NotImplementedError("write your pallas kernel here")


# speedup vs baseline: 5.8832x; 5.8832x over previous
import jax
import jax.numpy as jnp
from jax import lax
from jax.experimental import pallas as pl
from jax.experimental.pallas import tpu as pltpu
from jax.experimental.pallas import tpu_sc as plsc

N_NODES = 10000
N_PAD = 10240
N_EDGES = 320000
D_IN = 128
D_HID = 128
D_OUT = 40
D_OUT_PAD = 64

NC = 2
NS = 16
EDGES_PER_TILE = N_EDGES // NS
CHUNK = 80
NCHUNKS = EDGES_PER_TILE // CHUNK
ROWS_PER_TILE = N_PAD // NS
STAGE_ROWS = 128
CW = 8


def _make_sc_agg(d, with_cnt):
  dh = d // NC
  mesh = plsc.VectorSubcoreMesh(
      core_axis_name="c", subcore_axis_name="s", num_cores=NC,
      num_subcores=NS)

  out_type = [jax.ShapeDtypeStruct((NC, N_PAD, dh), jnp.float32)]
  scratch = [
      pltpu.VMEM((NCHUNKS, CHUNK), jnp.int32),
      pltpu.VMEM((NCHUNKS, CHUNK), jnp.int32),
      pltpu.VMEM((CHUNK, dh), jnp.float32),
      pltpu.VMEM((STAGE_ROWS, dh), jnp.float32),
      pltpu.VMEM_SHARED((N_PAD, dh), jnp.float32),
      pltpu.SemaphoreType.DMA,
  ]
  if with_cnt:
    out_type.append(jax.ShapeDtypeStruct((NC, N_PAD, CW), jnp.float32))
    scratch += [
        pltpu.VMEM((CHUNK, CW), jnp.float32),
        pltpu.VMEM((STAGE_ROWS, CW), jnp.float32),
        pltpu.VMEM_SHARED((N_PAD, CW), jnp.float32),
    ]

  def body(*refs):
    if with_cnt:
      (table_hbm, srcp_hbm, dst_hbm, zeros_hbm, ones_hbm, zcnt_hbm,
       out_hbm, cnt_hbm,
       srcv, dstv, rows, stage, acc_sh, sem,
       onesv, cstage, cacc_sh) = refs
    else:
      (table_hbm, srcp_hbm, dst_hbm, zeros_hbm,
       out_hbm,
       srcv, dstv, rows, stage, acc_sh, sem) = refs

    c = lax.axis_index("c")
    s = lax.axis_index("s")

    pltpu.sync_copy(zeros_hbm, stage)
    for k in range(ROWS_PER_TILE // STAGE_ROWS):
      base = s * ROWS_PER_TILE + k * STAGE_ROWS
      pltpu.sync_copy(stage, acc_sh.at[pl.ds(base, STAGE_ROWS)])
    if with_cnt:
      pltpu.sync_copy(ones_hbm, onesv)
      pltpu.sync_copy(zcnt_hbm, cstage)
      for k in range(ROWS_PER_TILE // STAGE_ROWS):
        base = s * ROWS_PER_TILE + k * STAGE_ROWS
        pltpu.sync_copy(cstage, cacc_sh.at[pl.ds(base, STAGE_ROWS)])

    pltpu.sync_copy(srcp_hbm.at[c, s], srcv)
    pltpu.sync_copy(dst_hbm.at[s], dstv)
    plsc.subcore_barrier()

    def step(j, carry):
      pltpu.async_copy(table_hbm.at[srcv.at[j]], rows, sem).wait()
      pltpu.sync_copy(rows, acc_sh.at[dstv.at[j]], add=True)
      if with_cnt:
        pltpu.sync_copy(onesv, cacc_sh.at[dstv.at[j]], add=True)
      return carry

    lax.fori_loop(0, NCHUNKS, step, 0)
    plsc.subcore_barrier()

    for k in range(ROWS_PER_TILE // STAGE_ROWS):
      base = s * ROWS_PER_TILE + k * STAGE_ROWS
      pltpu.sync_copy(acc_sh.at[pl.ds(base, STAGE_ROWS)], stage)
      pltpu.sync_copy(stage, out_hbm.at[c, pl.ds(base, STAGE_ROWS)])
      if with_cnt:
        pltpu.sync_copy(cacc_sh.at[pl.ds(base, STAGE_ROWS)], cstage)
        pltpu.sync_copy(cstage, cnt_hbm.at[c, pl.ds(base, STAGE_ROWS)])

  return pl.kernel(body, out_type=out_type, mesh=mesh,
                   scratch_types=scratch,
                   compiler_params=pltpu.CompilerParams(
                       use_tc_tiling_on_sc=False))


_sc_agg_l1 = _make_sc_agg(D_IN, with_cnt=True)
_sc_agg_l2 = _make_sc_agg(D_OUT_PAD, with_cnt=False)


ROW_BLK = 1000


def _tc1_body(s1_ref, cnt_ref, x_ref, w1l_ref, w1r_ref, b1_ref, w2l_ref,
              w2r_ref, b2_ref, hl_ref, hr_ref, inv_ref):
  cnt = cnt_ref[0, :, 0:1]
  inv = 1.0 / jnp.maximum(cnt, 1.0)
  inv_ref[...] = inv
  summ = jnp.concatenate([s1_ref[0], s1_ref[1]], axis=1)
  mean = summ * inv
  h = mean @ w1l_ref[...] + x_ref[...] @ w1r_ref[...] + b1_ref[...]
  h = jnp.maximum(h, 0.0)
  hl = h @ w2l_ref[...]
  hl_ref[0] = hl[:, : D_OUT_PAD // 2]
  hl_ref[1] = hl[:, D_OUT_PAD // 2 :]
  hr_ref[...] = h @ w2r_ref[...] + b2_ref[...]


def _tc2_body(s2_ref, inv_ref, hr_ref, out_ref):
  s2 = jnp.concatenate([s2_ref[0], s2_ref[1]], axis=1)
  out_ref[...] = s2 * inv_ref[...] + hr_ref[...]


@jax.jit
def kernel(x, edge_index, W1l, W1r, b1, W2l, W2r, b2):
  src = edge_index[0].reshape(NS, NCHUNKS, CHUNK)
  dst = edge_index[1].reshape(NS, NCHUNKS, CHUNK)
  srcp = jnp.stack([src, src + N_NODES])
  dh1 = D_IN // NC
  x2 = x.reshape(N_NODES, NC, dh1).transpose(1, 0, 2).reshape(
      NC * N_NODES, dh1)
  zeros_d1 = jnp.zeros((STAGE_ROWS, dh1), jnp.float32)
  zeros_dp = jnp.zeros((STAGE_ROWS, D_OUT_PAD // NC), jnp.float32)
  zeros_c = jnp.zeros((STAGE_ROWS, CW), jnp.float32)
  ones_c = jnp.ones((CHUNK, CW), jnp.float32)

  s1, cnt = _sc_agg_l1(x2, srcp, dst, zeros_d1, ones_c, zeros_c)

  w2l_pad = jnp.pad(W2l, ((0, 0), (0, D_OUT_PAD - D_OUT)))
  w2r_pad = jnp.pad(W2r, ((0, 0), (0, D_OUT_PAD - D_OUT)))
  b2_pad = jnp.pad(b2, (0, D_OUT_PAD - D_OUT)).reshape(1, D_OUT_PAD)
  b1_2d = b1.reshape(1, D_HID)
  grid = (N_NODES // ROW_BLK,)
  dh2 = D_OUT_PAD // NC
  hl2, hr, inv = pl.pallas_call(
      _tc1_body,
      grid=grid,
      in_specs=[
          pl.BlockSpec((NC, ROW_BLK, dh1), lambda i: (0, i, 0)),
          pl.BlockSpec((1, ROW_BLK, CW), lambda i: (0, i, 0)),
          pl.BlockSpec((ROW_BLK, D_IN), lambda i: (i, 0)),
          pl.BlockSpec((D_IN, D_HID), lambda i: (0, 0)),
          pl.BlockSpec((D_IN, D_HID), lambda i: (0, 0)),
          pl.BlockSpec((1, D_HID), lambda i: (0, 0)),
          pl.BlockSpec((D_HID, D_OUT_PAD), lambda i: (0, 0)),
          pl.BlockSpec((D_HID, D_OUT_PAD), lambda i: (0, 0)),
          pl.BlockSpec((1, D_OUT_PAD), lambda i: (0, 0)),
      ],
      out_specs=[
          pl.BlockSpec((NC, ROW_BLK, dh2), lambda i: (0, i, 0)),
          pl.BlockSpec((ROW_BLK, D_OUT_PAD), lambda i: (i, 0)),
          pl.BlockSpec((ROW_BLK, 1), lambda i: (i, 0)),
      ],
      out_shape=[
          jax.ShapeDtypeStruct((NC, N_NODES, dh2), jnp.float32),
          jax.ShapeDtypeStruct((N_NODES, D_OUT_PAD), jnp.float32),
          jax.ShapeDtypeStruct((N_NODES, 1), jnp.float32),
      ],
  )(s1, cnt, x, W1l, W1r, b1_2d, w2l_pad, w2r_pad, b2_pad)

  (s2,) = _sc_agg_l2(hl2.reshape(NC * N_NODES, dh2), srcp, dst, zeros_dp)

  out_pad = pl.pallas_call(
      _tc2_body,
      grid=grid,
      in_specs=[
          pl.BlockSpec((NC, ROW_BLK, dh2), lambda i: (0, i, 0)),
          pl.BlockSpec((ROW_BLK, 1), lambda i: (i, 0)),
          pl.BlockSpec((ROW_BLK, D_OUT_PAD), lambda i: (i, 0)),
      ],
      out_specs=pl.BlockSpec((ROW_BLK, D_OUT_PAD), lambda i: (i, 0)),
      out_shape=jax.ShapeDtypeStruct((N_NODES, D_OUT_PAD), jnp.float32),
  )(s2, inv, hr)

  return out_pad[:, :D_OUT]


# speedup vs baseline: 9.5361x; 1.6209x over previous
import jax
import jax.numpy as jnp
from jax import lax
from jax.experimental import pallas as pl
from jax.experimental.pallas import tpu as pltpu
from jax.experimental.pallas import tpu_sc as plsc

N_NODES = 10000
N_PAD = 10240
N_EDGES = 320000
D_IN = 128
D_HID = 128
D_OUT = 40
D_OUT_PAD = 64

NC = 2
NS = 16
EDGES_PER_TILE = N_EDGES // NS
CHUNK = 80
NCHUNKS = EDGES_PER_TILE // CHUNK
ROWS_PER_TILE = N_PAD // NS
STAGE_ROWS = 128
CW = 8


def _make_sc_agg(d, with_cnt):
  dh = d // NC
  mesh = plsc.VectorSubcoreMesh(
      core_axis_name="c", subcore_axis_name="s", num_cores=NC,
      num_subcores=NS)

  out_type = [jax.ShapeDtypeStruct((NC, N_PAD, dh), jnp.float32)]
  scratch = [
      pltpu.VMEM((NCHUNKS, CHUNK), jnp.int32),
      pltpu.VMEM((NCHUNKS, CHUNK), jnp.int32),
      pltpu.VMEM((CHUNK, dh), jnp.float32),
      pltpu.VMEM((CHUNK, dh), jnp.float32),
      pltpu.VMEM((STAGE_ROWS, dh), jnp.float32),
      pltpu.VMEM_SHARED((N_PAD, dh), jnp.float32),
      pltpu.SemaphoreType.DMA,
      pltpu.SemaphoreType.DMA,
  ]
  if with_cnt:
    out_type.append(jax.ShapeDtypeStruct((NC, N_PAD, CW), jnp.float32))
    scratch += [
        pltpu.VMEM((CHUNK, CW), jnp.float32),
        pltpu.VMEM((STAGE_ROWS, CW), jnp.float32),
        pltpu.VMEM_SHARED((N_PAD, CW), jnp.float32),
        pltpu.SemaphoreType.DMA,
    ]

  def body(*refs):
    if with_cnt:
      (table_hbm, srcp_hbm, dst_hbm, zeros_hbm, ones_hbm, zcnt_hbm,
       out_hbm, cnt_hbm,
       srcv, dstv, rows_a, rows_b, stage, acc_sh, sem_a, sem_b,
       onesv, cstage, cacc_sh, csem) = refs
    else:
      (table_hbm, srcp_hbm, dst_hbm, zeros_hbm,
       out_hbm,
       srcv, dstv, rows_a, rows_b, stage, acc_sh, sem_a, sem_b) = refs

    c = lax.axis_index("c")
    s = lax.axis_index("s")

    pltpu.sync_copy(zeros_hbm, stage)
    for k in range(ROWS_PER_TILE // STAGE_ROWS):
      base = s * ROWS_PER_TILE + k * STAGE_ROWS
      pltpu.sync_copy(stage, acc_sh.at[pl.ds(base, STAGE_ROWS)])
    if with_cnt:
      pltpu.sync_copy(ones_hbm, onesv)
      pltpu.sync_copy(zcnt_hbm, cstage)
      for k in range(ROWS_PER_TILE // STAGE_ROWS):
        base = s * ROWS_PER_TILE + k * STAGE_ROWS
        pltpu.sync_copy(cstage, cacc_sh.at[pl.ds(base, STAGE_ROWS)])

    pltpu.sync_copy(srcp_hbm.at[c, s], srcv)
    pltpu.sync_copy(dst_hbm.at[s], dstv)
    plsc.subcore_barrier()

    npairs = NCHUNKS // 2
    pltpu.async_copy(table_hbm.at[srcv.at[0]], rows_a, sem_a)

    def step(t, carry):
      j0 = 2 * t
      j1 = j0 + 1
      jn = jnp.minimum(j0 + 2, NCHUNKS - 1)
      pltpu.async_copy(table_hbm.at[srcv.at[j1]], rows_b, sem_b)
      if with_cnt:
        jc = c * npairs + t
        pltpu.async_copy(onesv, cacc_sh.at[dstv.at[jc]], csem, add=True)
      pltpu.make_async_copy(table_hbm.at[srcv.at[j0]], rows_a, sem_a).wait()
      pltpu.sync_copy(rows_a, acc_sh.at[dstv.at[j0]], add=True)
      pltpu.async_copy(table_hbm.at[srcv.at[jn]], rows_a, sem_a)
      pltpu.make_async_copy(table_hbm.at[srcv.at[j1]], rows_b, sem_b).wait()
      pltpu.sync_copy(rows_b, acc_sh.at[dstv.at[j1]], add=True)
      return carry

    lax.fori_loop(0, npairs, step, 0)
    pltpu.make_async_copy(table_hbm.at[srcv.at[0]], rows_a, sem_a).wait()
    if with_cnt:
      def drain(t, carry):
        pltpu.make_async_copy(onesv, cacc_sh.at[dstv.at[0]], csem).wait()
        return carry
      lax.fori_loop(0, npairs, drain, 0)
    plsc.subcore_barrier()

    for k in range(ROWS_PER_TILE // STAGE_ROWS):
      base = s * ROWS_PER_TILE + k * STAGE_ROWS
      pltpu.sync_copy(acc_sh.at[pl.ds(base, STAGE_ROWS)], stage)
      pltpu.sync_copy(stage, out_hbm.at[c, pl.ds(base, STAGE_ROWS)])
      if with_cnt:
        pltpu.sync_copy(cacc_sh.at[pl.ds(base, STAGE_ROWS)], cstage)
        pltpu.sync_copy(cstage, cnt_hbm.at[c, pl.ds(base, STAGE_ROWS)])

  return pl.kernel(body, out_type=out_type, mesh=mesh,
                   scratch_types=scratch,
                   compiler_params=pltpu.CompilerParams(
                       use_tc_tiling_on_sc=False))


_sc_agg_l1 = _make_sc_agg(D_IN, with_cnt=True)
_sc_agg_l2 = _make_sc_agg(D_OUT_PAD, with_cnt=False)


ROW_BLK = 1000


def _tc1_body(s1_ref, cnt_ref, x_ref, w1l_ref, w1r_ref, b1_ref, w2l_ref,
              w2r_ref, b2_ref, hl_ref, hr_ref, inv_ref):
  cnt = cnt_ref[0, :, 0:1] + cnt_ref[1, :, 0:1]
  inv = 1.0 / jnp.maximum(cnt, 1.0)
  inv_ref[...] = inv
  summ = jnp.concatenate([s1_ref[0], s1_ref[1]], axis=1)
  mean = summ * inv
  h = mean @ w1l_ref[...] + x_ref[...] @ w1r_ref[...] + b1_ref[...]
  h = jnp.maximum(h, 0.0)
  hl = h @ w2l_ref[...]
  hl_ref[0] = hl[:, : D_OUT_PAD // 2]
  hl_ref[1] = hl[:, D_OUT_PAD // 2 :]
  hr_ref[...] = h @ w2r_ref[...] + b2_ref[...]


def _tc2_body(s2_ref, inv_ref, hr_ref, out_ref):
  s2 = jnp.concatenate([s2_ref[0], s2_ref[1]], axis=1)
  out_ref[...] = s2 * inv_ref[...] + hr_ref[...]


@jax.jit
def kernel(x, edge_index, W1l, W1r, b1, W2l, W2r, b2):
  src = edge_index[0].reshape(NS, NCHUNKS, CHUNK)
  dst = edge_index[1].reshape(NS, NCHUNKS, CHUNK)
  srcp = jnp.stack([src, src + N_NODES])
  dh1 = D_IN // NC
  x2 = x.reshape(N_NODES, NC, dh1).transpose(1, 0, 2).reshape(
      NC * N_NODES, dh1)
  zeros_d1 = jnp.zeros((STAGE_ROWS, dh1), jnp.float32)
  zeros_dp = jnp.zeros((STAGE_ROWS, D_OUT_PAD // NC), jnp.float32)
  zeros_c = jnp.zeros((STAGE_ROWS, CW), jnp.float32)
  ones_c = jnp.ones((CHUNK, CW), jnp.float32)

  s1, cnt = _sc_agg_l1(x2, srcp, dst, zeros_d1, ones_c, zeros_c)

  w2l_pad = jnp.pad(W2l, ((0, 0), (0, D_OUT_PAD - D_OUT)))
  w2r_pad = jnp.pad(W2r, ((0, 0), (0, D_OUT_PAD - D_OUT)))
  b2_pad = jnp.pad(b2, (0, D_OUT_PAD - D_OUT)).reshape(1, D_OUT_PAD)
  b1_2d = b1.reshape(1, D_HID)
  grid = (N_NODES // ROW_BLK,)
  dh2 = D_OUT_PAD // NC
  hl2, hr, inv = pl.pallas_call(
      _tc1_body,
      grid=grid,
      in_specs=[
          pl.BlockSpec((NC, ROW_BLK, dh1), lambda i: (0, i, 0)),
          pl.BlockSpec((NC, ROW_BLK, CW), lambda i: (0, i, 0)),
          pl.BlockSpec((ROW_BLK, D_IN), lambda i: (i, 0)),
          pl.BlockSpec((D_IN, D_HID), lambda i: (0, 0)),
          pl.BlockSpec((D_IN, D_HID), lambda i: (0, 0)),
          pl.BlockSpec((1, D_HID), lambda i: (0, 0)),
          pl.BlockSpec((D_HID, D_OUT_PAD), lambda i: (0, 0)),
          pl.BlockSpec((D_HID, D_OUT_PAD), lambda i: (0, 0)),
          pl.BlockSpec((1, D_OUT_PAD), lambda i: (0, 0)),
      ],
      out_specs=[
          pl.BlockSpec((NC, ROW_BLK, dh2), lambda i: (0, i, 0)),
          pl.BlockSpec((ROW_BLK, D_OUT_PAD), lambda i: (i, 0)),
          pl.BlockSpec((ROW_BLK, 1), lambda i: (i, 0)),
      ],
      out_shape=[
          jax.ShapeDtypeStruct((NC, N_NODES, dh2), jnp.float32),
          jax.ShapeDtypeStruct((N_NODES, D_OUT_PAD), jnp.float32),
          jax.ShapeDtypeStruct((N_NODES, 1), jnp.float32),
      ],
  )(s1, cnt, x, W1l, W1r, b1_2d, w2l_pad, w2r_pad, b2_pad)

  (s2,) = _sc_agg_l2(hl2.reshape(NC * N_NODES, dh2), srcp, dst, zeros_dp)

  out_pad = pl.pallas_call(
      _tc2_body,
      grid=grid,
      in_specs=[
          pl.BlockSpec((NC, ROW_BLK, dh2), lambda i: (0, i, 0)),
          pl.BlockSpec((ROW_BLK, 1), lambda i: (i, 0)),
          pl.BlockSpec((ROW_BLK, D_OUT_PAD), lambda i: (i, 0)),
      ],
      out_specs=pl.BlockSpec((ROW_BLK, D_OUT_PAD), lambda i: (i, 0)),
      out_shape=jax.ShapeDtypeStruct((N_NODES, D_OUT_PAD), jnp.float32),
  )(s2, inv, hr)

  return out_pad[:, :D_OUT]


# speedup vs baseline: 12.1338x; 1.2724x over previous
import jax
import jax.numpy as jnp
from jax import lax
from jax.experimental import pallas as pl
from jax.experimental.pallas import tpu as pltpu
from jax.experimental.pallas import tpu_sc as plsc

N_NODES = 10000
N_PAD = 10240
N_EDGES = 320000
D_IN = 128
D_HID = 128
D_OUT = 40
D_OUT_PAD = 64

NC = 2
NS = 16
EDGES_PER_TILE = N_EDGES // NS
CHUNK = 100
NCHUNKS = EDGES_PER_TILE // CHUNK
NBUF = 4
ROUNDS = NCHUNKS // NBUF
CNT_PER_CORE = NCHUNKS // NC
CPR = CNT_PER_CORE // ROUNDS
ROWS_PER_TILE = N_PAD // NS
STAGE_ROWS = 128
CW = 8


def _make_sc_agg(d, with_cnt):
  dh = d // NC
  mesh = plsc.VectorSubcoreMesh(
      core_axis_name="c", subcore_axis_name="s", num_cores=NC,
      num_subcores=NS)

  out_type = [jax.ShapeDtypeStruct((NC, N_PAD, dh), jnp.float32)]
  scratch = [
      pltpu.VMEM((NCHUNKS, CHUNK), jnp.int32),
      pltpu.VMEM((NCHUNKS, CHUNK), jnp.int32),
      *[pltpu.VMEM((CHUNK, dh), jnp.float32) for _ in range(NBUF)],
      pltpu.VMEM((STAGE_ROWS, dh), jnp.float32),
      pltpu.VMEM_SHARED((N_PAD, dh), jnp.float32),
      *[pltpu.SemaphoreType.DMA for _ in range(2 * NBUF)],
  ]
  if with_cnt:
    out_type.append(jax.ShapeDtypeStruct((NC, N_PAD, CW), jnp.float32))
    scratch += [
        pltpu.VMEM((CHUNK, CW), jnp.float32),
        pltpu.VMEM((STAGE_ROWS, CW), jnp.float32),
        pltpu.VMEM_SHARED((N_PAD, CW), jnp.float32),
        pltpu.SemaphoreType.DMA,
    ]

  def body(*refs):
    if with_cnt:
      (table_hbm, srcp_hbm, dst_hbm, zeros_hbm, ones_hbm, zcnt_hbm,
       out_hbm, cnt_hbm,
       srcv, dstv, *rest) = refs
      rows = rest[:NBUF]
      stage, acc_sh = rest[NBUF], rest[NBUF + 1]
      gsem = rest[NBUF + 2:2 * NBUF + 2]
      ssem = rest[2 * NBUF + 2:3 * NBUF + 2]
      onesv, cstage, cacc_sh, csem = rest[3 * NBUF + 2:]
    else:
      (table_hbm, srcp_hbm, dst_hbm, zeros_hbm,
       out_hbm,
       srcv, dstv, *rest) = refs
      rows = rest[:NBUF]
      stage, acc_sh = rest[NBUF], rest[NBUF + 1]
      gsem = rest[NBUF + 2:2 * NBUF + 2]
      ssem = rest[2 * NBUF + 2:3 * NBUF + 2]

    c = lax.axis_index("c")
    s = lax.axis_index("s")

    pltpu.sync_copy(zeros_hbm, stage)
    for k in range(ROWS_PER_TILE // STAGE_ROWS):
      base = s * ROWS_PER_TILE + k * STAGE_ROWS
      pltpu.sync_copy(stage, acc_sh.at[pl.ds(base, STAGE_ROWS)])
    if with_cnt:
      pltpu.sync_copy(ones_hbm, onesv)
      pltpu.sync_copy(zcnt_hbm, cstage)
      for k in range(ROWS_PER_TILE // STAGE_ROWS):
        base = s * ROWS_PER_TILE + k * STAGE_ROWS
        pltpu.sync_copy(cstage, cacc_sh.at[pl.ds(base, STAGE_ROWS)])

    pltpu.sync_copy(srcp_hbm.at[c, s], srcv)
    pltpu.sync_copy(dst_hbm.at[s], dstv)
    plsc.subcore_barrier()

    for k in range(NBUF):
      pltpu.async_copy(table_hbm.at[srcv.at[k]], rows[k], gsem[k])

    def step(t, carry):
      j0 = NBUF * t
      for k in range(NBUF):
        pltpu.make_async_copy(
            table_hbm.at[srcv.at[0]], rows[k], gsem[k]).wait()
        pltpu.async_copy(rows[k], acc_sh.at[dstv.at[j0 + k]], ssem[k],
                         add=True)
      if with_cnt:
        for i in range(CPR):
          jc = c * CNT_PER_CORE + CPR * t + i
          pltpu.async_copy(onesv, cacc_sh.at[dstv.at[jc]], csem, add=True)
      for k in range(NBUF):
        jn = jnp.minimum(j0 + NBUF + k, NCHUNKS - 1)
        pltpu.make_async_copy(
            rows[k], acc_sh.at[dstv.at[0]], ssem[k]).wait()
        pltpu.async_copy(table_hbm.at[srcv.at[jn]], rows[k], gsem[k])
      return carry

    lax.fori_loop(0, ROUNDS, step, 0)
    for k in range(NBUF):
      pltpu.make_async_copy(table_hbm.at[srcv.at[0]], rows[k], gsem[k]).wait()
    if with_cnt:
      def drain(t, carry):
        pltpu.make_async_copy(onesv, cacc_sh.at[dstv.at[0]], csem).wait()
        return carry
      lax.fori_loop(0, CNT_PER_CORE, drain, 0)
    plsc.subcore_barrier()

    for k in range(ROWS_PER_TILE // STAGE_ROWS):
      base = s * ROWS_PER_TILE + k * STAGE_ROWS
      pltpu.sync_copy(acc_sh.at[pl.ds(base, STAGE_ROWS)], stage)
      pltpu.sync_copy(stage, out_hbm.at[c, pl.ds(base, STAGE_ROWS)])
      if with_cnt:
        pltpu.sync_copy(cacc_sh.at[pl.ds(base, STAGE_ROWS)], cstage)
        pltpu.sync_copy(cstage, cnt_hbm.at[c, pl.ds(base, STAGE_ROWS)])

  return pl.kernel(body, out_type=out_type, mesh=mesh,
                   scratch_types=scratch,
                   compiler_params=pltpu.CompilerParams(
                       use_tc_tiling_on_sc=False))


_sc_agg_l1 = _make_sc_agg(D_IN, with_cnt=True)
_sc_agg_l2 = _make_sc_agg(D_OUT_PAD, with_cnt=False)


ROW_BLK = 1000


def _tc1_body(s1_ref, cnt_ref, x_ref, w1l_ref, w1r_ref, b1_ref, w2l_ref,
              w2r_ref, b2_ref, hl_ref, hr_ref, inv_ref):
  cnt = cnt_ref[0, :, 0:1] + cnt_ref[1, :, 0:1]
  inv = 1.0 / jnp.maximum(cnt, 1.0)
  inv_ref[...] = inv
  summ = jnp.concatenate([s1_ref[0], s1_ref[1]], axis=1)
  mean = summ * inv
  h = mean @ w1l_ref[...] + x_ref[...] @ w1r_ref[...] + b1_ref[...]
  h = jnp.maximum(h, 0.0)
  hl = h @ w2l_ref[...]
  hl_ref[0] = hl[:, : D_OUT_PAD // 2]
  hl_ref[1] = hl[:, D_OUT_PAD // 2 :]
  hr_ref[...] = h @ w2r_ref[...] + b2_ref[...]


def _tc2_body(s2_ref, inv_ref, hr_ref, out_ref):
  s2 = jnp.concatenate([s2_ref[0], s2_ref[1]], axis=1)
  out_ref[...] = s2 * inv_ref[...] + hr_ref[...]


@jax.jit
def kernel(x, edge_index, W1l, W1r, b1, W2l, W2r, b2):
  src = edge_index[0].reshape(NS, NCHUNKS, CHUNK)
  dst = edge_index[1].reshape(NS, NCHUNKS, CHUNK)
  srcp = jnp.stack([src, src + N_NODES])
  dh1 = D_IN // NC
  x2 = x.reshape(N_NODES, NC, dh1).transpose(1, 0, 2).reshape(
      NC * N_NODES, dh1)
  zeros_d1 = jnp.zeros((STAGE_ROWS, dh1), jnp.float32)
  zeros_dp = jnp.zeros((STAGE_ROWS, D_OUT_PAD // NC), jnp.float32)
  zeros_c = jnp.zeros((STAGE_ROWS, CW), jnp.float32)
  ones_c = jnp.ones((CHUNK, CW), jnp.float32)

  s1, cnt = _sc_agg_l1(x2, srcp, dst, zeros_d1, ones_c, zeros_c)

  w2l_pad = jnp.pad(W2l, ((0, 0), (0, D_OUT_PAD - D_OUT)))
  w2r_pad = jnp.pad(W2r, ((0, 0), (0, D_OUT_PAD - D_OUT)))
  b2_pad = jnp.pad(b2, (0, D_OUT_PAD - D_OUT)).reshape(1, D_OUT_PAD)
  b1_2d = b1.reshape(1, D_HID)
  grid = (N_NODES // ROW_BLK,)
  dh2 = D_OUT_PAD // NC
  hl2, hr, inv = pl.pallas_call(
      _tc1_body,
      grid=grid,
      in_specs=[
          pl.BlockSpec((NC, ROW_BLK, dh1), lambda i: (0, i, 0)),
          pl.BlockSpec((NC, ROW_BLK, CW), lambda i: (0, i, 0)),
          pl.BlockSpec((ROW_BLK, D_IN), lambda i: (i, 0)),
          pl.BlockSpec((D_IN, D_HID), lambda i: (0, 0)),
          pl.BlockSpec((D_IN, D_HID), lambda i: (0, 0)),
          pl.BlockSpec((1, D_HID), lambda i: (0, 0)),
          pl.BlockSpec((D_HID, D_OUT_PAD), lambda i: (0, 0)),
          pl.BlockSpec((D_HID, D_OUT_PAD), lambda i: (0, 0)),
          pl.BlockSpec((1, D_OUT_PAD), lambda i: (0, 0)),
      ],
      out_specs=[
          pl.BlockSpec((NC, ROW_BLK, dh2), lambda i: (0, i, 0)),
          pl.BlockSpec((ROW_BLK, D_OUT_PAD), lambda i: (i, 0)),
          pl.BlockSpec((ROW_BLK, 1), lambda i: (i, 0)),
      ],
      out_shape=[
          jax.ShapeDtypeStruct((NC, N_NODES, dh2), jnp.float32),
          jax.ShapeDtypeStruct((N_NODES, D_OUT_PAD), jnp.float32),
          jax.ShapeDtypeStruct((N_NODES, 1), jnp.float32),
      ],
  )(s1, cnt, x, W1l, W1r, b1_2d, w2l_pad, w2r_pad, b2_pad)

  (s2,) = _sc_agg_l2(hl2.reshape(NC * N_NODES, dh2), srcp, dst, zeros_dp)

  out_pad = pl.pallas_call(
      _tc2_body,
      grid=grid,
      in_specs=[
          pl.BlockSpec((NC, ROW_BLK, dh2), lambda i: (0, i, 0)),
          pl.BlockSpec((ROW_BLK, 1), lambda i: (i, 0)),
          pl.BlockSpec((ROW_BLK, D_OUT_PAD), lambda i: (i, 0)),
      ],
      out_specs=pl.BlockSpec((ROW_BLK, D_OUT_PAD), lambda i: (i, 0)),
      out_shape=jax.ShapeDtypeStruct((N_NODES, D_OUT_PAD), jnp.float32),
  )(s2, inv, hr)

  return out_pad[:, :D_OUT]


# speedup vs baseline: 13.0462x; 1.0752x over previous
import jax
import jax.numpy as jnp
from jax import lax
from jax.experimental import pallas as pl
from jax.experimental.pallas import tpu as pltpu
from jax.experimental.pallas import tpu_sc as plsc

N_NODES = 10000
N_PAD = 10240
N_EDGES = 320000
D_IN = 128
D_HID = 128
D_OUT = 40
D_OUT_PAD = 64

NC = 2
NS = 16
EDGES_PER_TILE = N_EDGES // NS
CHUNK = 100
NCHUNKS = EDGES_PER_TILE // CHUNK
NBUF = 4
ROUNDS = NCHUNKS // NBUF
CNT_PER_CORE = NCHUNKS // NC
CPR = CNT_PER_CORE // ROUNDS
ROWS_PER_TILE = N_PAD // NS
STAGE_ROWS = 128
CW = 8


def _make_sc_agg(d, with_cnt):
  dh = d // NC
  mesh = plsc.VectorSubcoreMesh(
      core_axis_name="c", subcore_axis_name="s", num_cores=NC,
      num_subcores=NS)

  out_type = [jax.ShapeDtypeStruct((NC, N_PAD, dh), jnp.float32)]
  scratch = [
      pltpu.VMEM((NCHUNKS, CHUNK), jnp.int32),
      pltpu.VMEM((NCHUNKS, CHUNK), jnp.int32),
      *[pltpu.VMEM((CHUNK, dh), jnp.float32) for _ in range(NBUF)],
      pltpu.VMEM((STAGE_ROWS, dh), jnp.float32),
      pltpu.VMEM_SHARED((N_PAD, dh), jnp.float32),
      *[pltpu.SemaphoreType.DMA for _ in range(2 * NBUF)],
  ]
  if with_cnt:
    out_type.append(jax.ShapeDtypeStruct((NC, N_PAD, CW), jnp.float32))
    scratch += [
        pltpu.VMEM((CHUNK, CW), jnp.float32),
        pltpu.VMEM((STAGE_ROWS, CW), jnp.float32),
        pltpu.VMEM_SHARED((N_PAD, CW), jnp.float32),
        pltpu.SemaphoreType.DMA,
    ]

  def body(*refs):
    if with_cnt:
      (table_hbm, srcp_hbm, dst_hbm, zeros_hbm, ones_hbm, zcnt_hbm,
       out_hbm, cnt_hbm,
       srcv, dstv, *rest) = refs
      rows = rest[:NBUF]
      stage, acc_sh = rest[NBUF], rest[NBUF + 1]
      gsem = rest[NBUF + 2:2 * NBUF + 2]
      ssem = rest[2 * NBUF + 2:3 * NBUF + 2]
      onesv, cstage, cacc_sh, csem = rest[3 * NBUF + 2:]
    else:
      (table_hbm, srcp_hbm, dst_hbm, zeros_hbm,
       out_hbm,
       srcv, dstv, *rest) = refs
      rows = rest[:NBUF]
      stage, acc_sh = rest[NBUF], rest[NBUF + 1]
      gsem = rest[NBUF + 2:2 * NBUF + 2]
      ssem = rest[2 * NBUF + 2:3 * NBUF + 2]

    c = lax.axis_index("c")
    s = lax.axis_index("s")

    pltpu.sync_copy(zeros_hbm, stage)
    for k in range(ROWS_PER_TILE // STAGE_ROWS):
      base = s * ROWS_PER_TILE + k * STAGE_ROWS
      pltpu.sync_copy(stage, acc_sh.at[pl.ds(base, STAGE_ROWS)])
    if with_cnt:
      pltpu.sync_copy(ones_hbm, onesv)
      pltpu.sync_copy(zcnt_hbm, cstage)
      for k in range(ROWS_PER_TILE // STAGE_ROWS):
        base = s * ROWS_PER_TILE + k * STAGE_ROWS
        pltpu.sync_copy(cstage, cacc_sh.at[pl.ds(base, STAGE_ROWS)])

    pltpu.sync_copy(srcp_hbm.at[c, s], srcv)
    pltpu.sync_copy(dst_hbm.at[s], dstv)
    plsc.subcore_barrier()

    for k in range(NBUF):
      pltpu.async_copy(table_hbm.at[srcv.at[k]], rows[k], gsem[k])

    def step(t, carry):
      j0 = NBUF * t
      for k in range(NBUF):
        pltpu.make_async_copy(
            table_hbm.at[srcv.at[0]], rows[k], gsem[k]).wait()
        pltpu.async_copy(rows[k], acc_sh.at[dstv.at[j0 + k]], ssem[k],
                         add=True)
      if with_cnt:
        for i in range(CPR):
          jc = c * CNT_PER_CORE + CPR * t + i
          pltpu.async_copy(onesv, cacc_sh.at[dstv.at[jc]], csem, add=True)
      for k in range(NBUF):
        jn = jnp.minimum(j0 + NBUF + k, NCHUNKS - 1)
        pltpu.make_async_copy(
            rows[k], acc_sh.at[dstv.at[0]], ssem[k]).wait()
        pltpu.async_copy(table_hbm.at[srcv.at[jn]], rows[k], gsem[k])
      return carry

    lax.fori_loop(0, ROUNDS, step, 0)
    for k in range(NBUF):
      pltpu.make_async_copy(table_hbm.at[srcv.at[0]], rows[k], gsem[k]).wait()
    if with_cnt:
      def drain(t, carry):
        pltpu.make_async_copy(onesv, cacc_sh.at[dstv.at[0]], csem).wait()
        return carry
      lax.fori_loop(0, CNT_PER_CORE, drain, 0)
    plsc.subcore_barrier()

    for k in range(ROWS_PER_TILE // STAGE_ROWS):
      base = s * ROWS_PER_TILE + k * STAGE_ROWS
      pltpu.sync_copy(acc_sh.at[pl.ds(base, STAGE_ROWS)], stage)
      pltpu.sync_copy(stage, out_hbm.at[c, pl.ds(base, STAGE_ROWS)])
      if with_cnt:
        pltpu.sync_copy(cacc_sh.at[pl.ds(base, STAGE_ROWS)], cstage)
        pltpu.sync_copy(cstage, cnt_hbm.at[c, pl.ds(base, STAGE_ROWS)])

  return pl.kernel(body, out_type=out_type, mesh=mesh,
                   scratch_types=scratch,
                   compiler_params=pltpu.CompilerParams(
                       use_tc_tiling_on_sc=False))


_sc_agg_l1 = _make_sc_agg(D_IN, with_cnt=True)
_sc_agg_l2 = _make_sc_agg(D_OUT_PAD, with_cnt=False)


ROW_BLK = 1000


def _tc1_body(s1_ref, cnt_ref, x_ref, w1l_ref, w1r_ref, b1_ref, w2l_ref,
              w2r_ref, b2_ref, hl_ref, hr_ref, inv_ref):
  cnt = cnt_ref[0, :, 0:1] + cnt_ref[1, :, 0:1]
  inv = 1.0 / jnp.maximum(cnt, 1.0)
  inv_ref[...] = inv
  summ = jnp.concatenate([s1_ref[0], s1_ref[1]], axis=1)
  mean = summ * inv
  h = mean @ w1l_ref[...] + x_ref[...] @ w1r_ref[...] + b1_ref[...]
  h = jnp.maximum(h, 0.0)
  hl_ref[...] = h @ w2l_ref[...]
  hr_ref[...] = h @ w2r_ref[...] + b2_ref[...]


def _tc2_body(s2_ref, inv_ref, hr_ref, out_ref):
  s2 = jnp.concatenate([s2_ref[0], s2_ref[1]], axis=1)
  out_ref[...] = (s2 * inv_ref[...] + hr_ref[...])[:, :D_OUT]


@jax.jit
def kernel(x, edge_index, W1l, W1r, b1, W2l, W2r, b2):
  src2 = (edge_index[0] * 2).reshape(NS, NCHUNKS, CHUNK)
  dst = edge_index[1].reshape(NS, NCHUNKS, CHUNK)
  srcp = jnp.stack([src2, src2 + 1])
  dh1 = D_IN // NC
  x2 = x.reshape(NC * N_NODES, dh1)
  zeros_d1 = jnp.zeros((STAGE_ROWS, dh1), jnp.float32)
  zeros_dp = jnp.zeros((STAGE_ROWS, D_OUT_PAD // NC), jnp.float32)
  zeros_c = jnp.zeros((STAGE_ROWS, CW), jnp.float32)
  ones_c = jnp.ones((CHUNK, CW), jnp.float32)

  s1, cnt = _sc_agg_l1(x2, srcp, dst, zeros_d1, ones_c, zeros_c)

  w2l_pad = jnp.pad(W2l, ((0, 0), (0, D_OUT_PAD - D_OUT)))
  w2r_pad = jnp.pad(W2r, ((0, 0), (0, D_OUT_PAD - D_OUT)))
  b2_pad = jnp.pad(b2, (0, D_OUT_PAD - D_OUT)).reshape(1, D_OUT_PAD)
  b1_2d = b1.reshape(1, D_HID)
  grid = (N_NODES // ROW_BLK,)
  dh2 = D_OUT_PAD // NC
  hl2, hr, inv = pl.pallas_call(
      _tc1_body,
      grid=grid,
      in_specs=[
          pl.BlockSpec((NC, ROW_BLK, dh1), lambda i: (0, i, 0)),
          pl.BlockSpec((NC, ROW_BLK, CW), lambda i: (0, i, 0)),
          pl.BlockSpec((ROW_BLK, D_IN), lambda i: (i, 0)),
          pl.BlockSpec((D_IN, D_HID), lambda i: (0, 0)),
          pl.BlockSpec((D_IN, D_HID), lambda i: (0, 0)),
          pl.BlockSpec((1, D_HID), lambda i: (0, 0)),
          pl.BlockSpec((D_HID, D_OUT_PAD), lambda i: (0, 0)),
          pl.BlockSpec((D_HID, D_OUT_PAD), lambda i: (0, 0)),
          pl.BlockSpec((1, D_OUT_PAD), lambda i: (0, 0)),
      ],
      out_specs=[
          pl.BlockSpec((ROW_BLK, D_OUT_PAD), lambda i: (i, 0)),
          pl.BlockSpec((ROW_BLK, D_OUT_PAD), lambda i: (i, 0)),
          pl.BlockSpec((ROW_BLK, 1), lambda i: (i, 0)),
      ],
      out_shape=[
          jax.ShapeDtypeStruct((N_NODES, D_OUT_PAD), jnp.float32),
          jax.ShapeDtypeStruct((N_NODES, D_OUT_PAD), jnp.float32),
          jax.ShapeDtypeStruct((N_NODES, 1), jnp.float32),
      ],
  )(s1, cnt, x, W1l, W1r, b1_2d, w2l_pad, w2r_pad, b2_pad)

  (s2,) = _sc_agg_l2(hl2.reshape(NC * N_NODES, dh2), srcp, dst, zeros_dp)

  out = pl.pallas_call(
      _tc2_body,
      grid=grid,
      in_specs=[
          pl.BlockSpec((NC, ROW_BLK, dh2), lambda i: (0, i, 0)),
          pl.BlockSpec((ROW_BLK, 1), lambda i: (i, 0)),
          pl.BlockSpec((ROW_BLK, D_OUT_PAD), lambda i: (i, 0)),
      ],
      out_specs=pl.BlockSpec((ROW_BLK, D_OUT), lambda i: (i, 0)),
      out_shape=jax.ShapeDtypeStruct((N_NODES, D_OUT), jnp.float32),
  )(s2, inv, hr)

  return out


# speedup vs baseline: 14.2012x; 1.0885x over previous
import jax
import jax.numpy as jnp
from jax import lax
from jax.experimental import pallas as pl
from jax.experimental.pallas import tpu as pltpu
from jax.experimental.pallas import tpu_sc as plsc

N_NODES = 10000
N_PAD = 10240
N_EDGES = 320000
D_IN = 128
D_HID = 128
D_OUT = 40
D_OUT_PAD = 64

NC = 2
NS = 16
EDGES_PER_TILE = N_EDGES // NS
CHUNK = 80
NCHUNKS = EDGES_PER_TILE // CHUNK
NBUF = 5
ROUNDS = NCHUNKS // NBUF
CNT_PER_CORE = NCHUNKS // NC
ROWS_PER_TILE = N_PAD // NS
STAGE_ROWS = 128
CW = 8


def _make_sc_agg(d, with_cnt):
  dh = d // NC
  mesh = plsc.VectorSubcoreMesh(
      core_axis_name="c", subcore_axis_name="s", num_cores=NC,
      num_subcores=NS)

  out_type = [jax.ShapeDtypeStruct((NC, N_PAD, dh), jnp.float32)]
  scratch = [
      pltpu.VMEM((NCHUNKS, CHUNK), jnp.int32),
      pltpu.VMEM((NCHUNKS, CHUNK), jnp.int32),
      *[pltpu.VMEM((CHUNK, dh), jnp.float32) for _ in range(NBUF)],
      pltpu.VMEM((STAGE_ROWS, dh), jnp.float32),
      pltpu.VMEM_SHARED((N_PAD, dh), jnp.float32),
      *[pltpu.SemaphoreType.DMA for _ in range(2 * NBUF)],
  ]
  if with_cnt:
    out_type.append(jax.ShapeDtypeStruct((NC, N_PAD, CW), jnp.float32))
    scratch += [
        pltpu.VMEM((CHUNK, CW), jnp.float32),
        pltpu.VMEM((STAGE_ROWS, CW), jnp.float32),
        pltpu.VMEM_SHARED((N_PAD, CW), jnp.float32),
        pltpu.SemaphoreType.DMA,
    ]

  def body(*refs):
    if with_cnt:
      (table_hbm, edges_hbm, zeros_hbm, ones_hbm, zcnt_hbm,
       out_hbm, cnt_hbm,
       srcv, dstv, *rest) = refs
      rows = rest[:NBUF]
      stage, acc_sh = rest[NBUF], rest[NBUF + 1]
      gsem = rest[NBUF + 2:2 * NBUF + 2]
      ssem = rest[2 * NBUF + 2:3 * NBUF + 2]
      onesv, cstage, cacc_sh, csem = rest[3 * NBUF + 2:]
    else:
      (table_hbm, edges_hbm, zeros_hbm,
       out_hbm,
       srcv, dstv, *rest) = refs
      rows = rest[:NBUF]
      stage, acc_sh = rest[NBUF], rest[NBUF + 1]
      gsem = rest[NBUF + 2:2 * NBUF + 2]
      ssem = rest[2 * NBUF + 2:3 * NBUF + 2]

    c = lax.axis_index("c")
    s = lax.axis_index("s")

    pltpu.sync_copy(zeros_hbm, stage)
    for k in range(ROWS_PER_TILE // STAGE_ROWS):
      base = s * ROWS_PER_TILE + k * STAGE_ROWS
      pltpu.sync_copy(stage, acc_sh.at[pl.ds(base, STAGE_ROWS)])
    if with_cnt:
      pltpu.sync_copy(ones_hbm, onesv)
      pltpu.sync_copy(zcnt_hbm, cstage)
      for k in range(ROWS_PER_TILE // STAGE_ROWS):
        base = s * ROWS_PER_TILE + k * STAGE_ROWS
        pltpu.sync_copy(cstage, cacc_sh.at[pl.ds(base, STAGE_ROWS)])

    pltpu.sync_copy(edges_hbm.at[0, s], srcv)
    pltpu.sync_copy(edges_hbm.at[1, s], dstv)

    def xform(j, carry):
      for k in range(CHUNK // 16):
        v = srcv[j, pl.ds(16 * k, 16)]
        srcv[j, pl.ds(16 * k, 16)] = v + v + c
      return carry

    lax.fori_loop(0, NCHUNKS, xform, 0)
    plsc.subcore_barrier()

    for k in range(NBUF):
      pltpu.async_copy(table_hbm.at[srcv.at[k]], rows[k], gsem[k])

    def step(t, carry):
      j0 = NBUF * t
      for k in range(NBUF):
        pltpu.make_async_copy(
            table_hbm.at[srcv.at[0]], rows[k], gsem[k]).wait()
        pltpu.async_copy(rows[k], acc_sh.at[dstv.at[j0 + k]], ssem[k],
                         add=True)
      for k in range(NBUF):
        jn = jnp.minimum(j0 + NBUF + k, NCHUNKS - 1)
        pltpu.make_async_copy(
            rows[k], acc_sh.at[dstv.at[0]], ssem[k]).wait()
        pltpu.async_copy(table_hbm.at[srcv.at[jn]], rows[k], gsem[k])
      return carry

    lax.fori_loop(0, ROUNDS, step, 0)
    if with_cnt:
      def cnt_fire(t, carry):
        jc = c * CNT_PER_CORE + t
        pltpu.async_copy(onesv, cacc_sh.at[dstv.at[jc]], csem, add=True)
        return carry
      lax.fori_loop(0, CNT_PER_CORE, cnt_fire, 0)
    for k in range(NBUF):
      pltpu.make_async_copy(table_hbm.at[srcv.at[0]], rows[k], gsem[k]).wait()
    if with_cnt:
      def drain(t, carry):
        pltpu.make_async_copy(onesv, cacc_sh.at[dstv.at[0]], csem).wait()
        return carry
      lax.fori_loop(0, CNT_PER_CORE, drain, 0)
    plsc.subcore_barrier()

    for k in range(ROWS_PER_TILE // STAGE_ROWS):
      base = s * ROWS_PER_TILE + k * STAGE_ROWS
      pltpu.sync_copy(acc_sh.at[pl.ds(base, STAGE_ROWS)], stage)
      pltpu.sync_copy(stage, out_hbm.at[c, pl.ds(base, STAGE_ROWS)])
      if with_cnt:
        pltpu.sync_copy(cacc_sh.at[pl.ds(base, STAGE_ROWS)], cstage)
        pltpu.sync_copy(cstage, cnt_hbm.at[c, pl.ds(base, STAGE_ROWS)])

  return pl.kernel(body, out_type=out_type, mesh=mesh,
                   scratch_types=scratch,
                   compiler_params=pltpu.CompilerParams(
                       use_tc_tiling_on_sc=False))


_sc_agg_l1 = _make_sc_agg(D_IN, with_cnt=True)
_sc_agg_l2 = _make_sc_agg(D_OUT_PAD, with_cnt=False)


ROW_BLK = 1000


def _tc1_body(s1_ref, cnt_ref, x_ref, w1l_ref, w1r_ref, b1_ref, w2l_ref,
              w2r_ref, b2_ref, hl_ref, hr_ref, inv_ref):
  cnt = cnt_ref[0, :, 0:1] + cnt_ref[1, :, 0:1]
  inv = 1.0 / jnp.maximum(cnt, 1.0)
  inv_ref[...] = inv
  summ = jnp.concatenate([s1_ref[0], s1_ref[1]], axis=1)
  mean = summ * inv
  h = mean @ w1l_ref[...] + x_ref[...] @ w1r_ref[...] + b1_ref[...]
  h = jnp.maximum(h, 0.0)
  hl_ref[...] = h @ w2l_ref[...]
  hr_ref[...] = h @ w2r_ref[...] + b2_ref[...]


def _tc2_body(s2_ref, inv_ref, hr_ref, out_ref):
  s2 = jnp.concatenate([s2_ref[0], s2_ref[1]], axis=1)
  out_ref[...] = (s2 * inv_ref[...] + hr_ref[...])[:, :D_OUT]


@jax.jit
def kernel(x, edge_index, W1l, W1r, b1, W2l, W2r, b2):
  edges = edge_index.reshape(2, NS, NCHUNKS, CHUNK)
  dh1 = D_IN // NC
  x2 = x.reshape(NC * N_NODES, dh1)
  zeros_d1 = jnp.zeros((STAGE_ROWS, dh1), jnp.float32)
  zeros_dp = jnp.zeros((STAGE_ROWS, D_OUT_PAD // NC), jnp.float32)
  zeros_c = jnp.zeros((STAGE_ROWS, CW), jnp.float32)
  ones_c = jnp.ones((CHUNK, CW), jnp.float32)

  s1, cnt = _sc_agg_l1(x2, edges, zeros_d1, ones_c, zeros_c)

  w2l_pad = jnp.pad(W2l, ((0, 0), (0, D_OUT_PAD - D_OUT)))
  w2r_pad = jnp.pad(W2r, ((0, 0), (0, D_OUT_PAD - D_OUT)))
  b2_pad = jnp.pad(b2, (0, D_OUT_PAD - D_OUT)).reshape(1, D_OUT_PAD)
  b1_2d = b1.reshape(1, D_HID)
  grid = (N_NODES // ROW_BLK,)
  dh2 = D_OUT_PAD // NC
  hl2, hr, inv = pl.pallas_call(
      _tc1_body,
      grid=grid,
      in_specs=[
          pl.BlockSpec((NC, ROW_BLK, dh1), lambda i: (0, i, 0)),
          pl.BlockSpec((NC, ROW_BLK, CW), lambda i: (0, i, 0)),
          pl.BlockSpec((ROW_BLK, D_IN), lambda i: (i, 0)),
          pl.BlockSpec((D_IN, D_HID), lambda i: (0, 0)),
          pl.BlockSpec((D_IN, D_HID), lambda i: (0, 0)),
          pl.BlockSpec((1, D_HID), lambda i: (0, 0)),
          pl.BlockSpec((D_HID, D_OUT_PAD), lambda i: (0, 0)),
          pl.BlockSpec((D_HID, D_OUT_PAD), lambda i: (0, 0)),
          pl.BlockSpec((1, D_OUT_PAD), lambda i: (0, 0)),
      ],
      out_specs=[
          pl.BlockSpec((ROW_BLK, D_OUT_PAD), lambda i: (i, 0)),
          pl.BlockSpec((ROW_BLK, D_OUT_PAD), lambda i: (i, 0)),
          pl.BlockSpec((ROW_BLK, 1), lambda i: (i, 0)),
      ],
      out_shape=[
          jax.ShapeDtypeStruct((N_NODES, D_OUT_PAD), jnp.float32),
          jax.ShapeDtypeStruct((N_NODES, D_OUT_PAD), jnp.float32),
          jax.ShapeDtypeStruct((N_NODES, 1), jnp.float32),
      ],
  )(s1, cnt, x, W1l, W1r, b1_2d, w2l_pad, w2r_pad, b2_pad)

  (s2,) = _sc_agg_l2(hl2.reshape(NC * N_NODES, dh2), edges, zeros_dp)

  out = pl.pallas_call(
      _tc2_body,
      grid=grid,
      in_specs=[
          pl.BlockSpec((NC, ROW_BLK, dh2), lambda i: (0, i, 0)),
          pl.BlockSpec((ROW_BLK, 1), lambda i: (i, 0)),
          pl.BlockSpec((ROW_BLK, D_OUT_PAD), lambda i: (i, 0)),
      ],
      out_specs=pl.BlockSpec((ROW_BLK, D_OUT), lambda i: (i, 0)),
      out_shape=jax.ShapeDtypeStruct((N_NODES, D_OUT), jnp.float32),
  )(s2, inv, hr)

  return out


# speedup vs baseline: 14.2485x; 1.0033x over previous
import jax
import jax.numpy as jnp
from jax import lax
from jax.experimental import pallas as pl
from jax.experimental.pallas import tpu as pltpu
from jax.experimental.pallas import tpu_sc as plsc

N_NODES = 10000
N_PAD = 10240
N_EDGES = 320000
D_IN = 128
D_HID = 128
D_OUT = 40
D_OUT_PAD = 64

NC = 2
NS = 16
EDGES_PER_TILE = N_EDGES // NS
CHUNK = 80
NCHUNKS = EDGES_PER_TILE // CHUNK
NBUF = 5
ROUNDS = NCHUNKS // NBUF
CNT_PER_CORE = NCHUNKS // NC
ROWS_PER_TILE = N_PAD // NS
STAGE_ROWS = 128
CW = 8


def _make_sc_agg(d, with_cnt):
  dh = d // NC
  mesh = plsc.VectorSubcoreMesh(
      core_axis_name="c", subcore_axis_name="s", num_cores=NC,
      num_subcores=NS)

  out_type = [jax.ShapeDtypeStruct((NC, N_PAD, dh), jnp.float32)]
  scratch = [
      pltpu.VMEM((NCHUNKS, CHUNK), jnp.int32),
      pltpu.VMEM((NCHUNKS, CHUNK), jnp.int32),
      *[pltpu.VMEM((CHUNK, dh), jnp.float32) for _ in range(NBUF)],
      pltpu.VMEM((STAGE_ROWS, dh), jnp.float32),
      pltpu.VMEM_SHARED((N_PAD, dh), jnp.float32),
      *[pltpu.SemaphoreType.DMA for _ in range(2 * NBUF)],
  ]
  if with_cnt:
    out_type.append(jax.ShapeDtypeStruct((NC, N_PAD, CW), jnp.float32))
    scratch += [
        pltpu.VMEM((CHUNK, CW), jnp.float32),
        pltpu.VMEM((STAGE_ROWS, CW), jnp.float32),
        pltpu.VMEM_SHARED((N_PAD, CW), jnp.float32),
        pltpu.SemaphoreType.DMA,
    ]

  def body(*refs):
    if with_cnt:
      (table_hbm, edges_hbm, zeros_hbm, ones_hbm, zcnt_hbm,
       out_hbm, cnt_hbm,
       srcv, dstv, *rest) = refs
      rows = rest[:NBUF]
      stage, acc_sh = rest[NBUF], rest[NBUF + 1]
      gsem = rest[NBUF + 2:2 * NBUF + 2]
      ssem = rest[2 * NBUF + 2:3 * NBUF + 2]
      onesv, cstage, cacc_sh, csem = rest[3 * NBUF + 2:]
    else:
      (table_hbm, edges_hbm, zeros_hbm,
       out_hbm,
       srcv, dstv, *rest) = refs
      rows = rest[:NBUF]
      stage, acc_sh = rest[NBUF], rest[NBUF + 1]
      gsem = rest[NBUF + 2:2 * NBUF + 2]
      ssem = rest[2 * NBUF + 2:3 * NBUF + 2]

    c = lax.axis_index("c")
    s = lax.axis_index("s")

    pltpu.sync_copy(zeros_hbm, stage)
    for k in range(ROWS_PER_TILE // STAGE_ROWS):
      base = s * ROWS_PER_TILE + k * STAGE_ROWS
      pltpu.sync_copy(stage, acc_sh.at[pl.ds(base, STAGE_ROWS)])
    if with_cnt:
      pltpu.sync_copy(ones_hbm, onesv)
      pltpu.sync_copy(zcnt_hbm, cstage)
      for k in range(ROWS_PER_TILE // STAGE_ROWS):
        base = s * ROWS_PER_TILE + k * STAGE_ROWS
        pltpu.sync_copy(cstage, cacc_sh.at[pl.ds(base, STAGE_ROWS)])

    pltpu.sync_copy(edges_hbm.at[0, s], srcv)
    pltpu.sync_copy(edges_hbm.at[1, s], dstv)

    def xform(j, carry):
      for k in range(CHUNK // 16):
        v = srcv[j, pl.ds(16 * k, 16)]
        srcv[j, pl.ds(16 * k, 16)] = v + v + c
      return carry

    lax.fori_loop(0, NCHUNKS, xform, 0)
    plsc.subcore_barrier()

    for k in range(NBUF):
      pltpu.async_copy(table_hbm.at[srcv.at[k]], rows[k], gsem[k])

    def step(t, carry):
      j0 = NBUF * t
      for k in range(NBUF):
        pltpu.make_async_copy(
            table_hbm.at[srcv.at[0]], rows[k], gsem[k]).wait()
        pltpu.async_copy(rows[k], acc_sh.at[dstv.at[j0 + k]], ssem[k],
                         add=True)
      for k in range(NBUF):
        jn = jnp.minimum(j0 + NBUF + k, NCHUNKS - 1)
        pltpu.make_async_copy(
            rows[k], acc_sh.at[dstv.at[0]], ssem[k]).wait()
        pltpu.async_copy(table_hbm.at[srcv.at[jn]], rows[k], gsem[k])
      return carry

    lax.fori_loop(0, ROUNDS, step, 0)
    if with_cnt:
      def cnt_fire(t, carry):
        jc = c * CNT_PER_CORE + t
        pltpu.async_copy(onesv, cacc_sh.at[dstv.at[jc]], csem, add=True)
        return carry
      lax.fori_loop(0, CNT_PER_CORE, cnt_fire, 0)
    for k in range(NBUF):
      pltpu.make_async_copy(table_hbm.at[srcv.at[0]], rows[k], gsem[k]).wait()
    if with_cnt:
      def drain(t, carry):
        pltpu.make_async_copy(onesv, cacc_sh.at[dstv.at[0]], csem).wait()
        return carry
      lax.fori_loop(0, CNT_PER_CORE, drain, 0)
    plsc.subcore_barrier()

    for k in range(ROWS_PER_TILE // STAGE_ROWS):
      base = s * ROWS_PER_TILE + k * STAGE_ROWS
      pltpu.sync_copy(acc_sh.at[pl.ds(base, STAGE_ROWS)], stage)
      pltpu.sync_copy(stage, out_hbm.at[c, pl.ds(base, STAGE_ROWS)])
      if with_cnt:
        pltpu.sync_copy(cacc_sh.at[pl.ds(base, STAGE_ROWS)], cstage)
        pltpu.sync_copy(cstage, cnt_hbm.at[c, pl.ds(base, STAGE_ROWS)])

  return pl.kernel(body, out_type=out_type, mesh=mesh,
                   scratch_types=scratch,
                   compiler_params=pltpu.CompilerParams(
                       use_tc_tiling_on_sc=False))


_sc_agg_l1 = _make_sc_agg(D_IN, with_cnt=True)
_sc_agg_l2 = _make_sc_agg(D_OUT_PAD, with_cnt=False)


ROW_BLK = 2000


def _tc0_body(x_ref, w1r_ref, b1_ref, xr_ref):
  xr_ref[...] = x_ref[...] @ w1r_ref[...] + b1_ref[...]


def _tc1_body(s1_ref, cnt_ref, xr_ref, w1l_ref, w2l_ref,
              w2r_ref, b2_ref, hl_ref, hr_ref, inv_ref):
  cnt = cnt_ref[0, :, 0:1] + cnt_ref[1, :, 0:1]
  inv = 1.0 / jnp.maximum(cnt, 1.0)
  inv_ref[...] = inv
  summ = jnp.concatenate([s1_ref[0], s1_ref[1]], axis=1)
  mean = summ * inv
  h = jnp.maximum(mean @ w1l_ref[...] + xr_ref[...], 0.0)
  hl_ref[...] = h @ w2l_ref[...]
  hr_ref[...] = h @ w2r_ref[...] + b2_ref[...]


def _tc2_body(s2_ref, inv_ref, hr_ref, out_ref):
  s2 = jnp.concatenate([s2_ref[0], s2_ref[1]], axis=1)
  out_ref[...] = (s2 * inv_ref[...] + hr_ref[...])[:, :D_OUT]


@jax.jit
def kernel(x, edge_index, W1l, W1r, b1, W2l, W2r, b2):
  edges = edge_index.reshape(2, NS, NCHUNKS, CHUNK)
  dh1 = D_IN // NC
  x2 = x.reshape(NC * N_NODES, dh1)
  zeros_d1 = jnp.zeros((STAGE_ROWS, dh1), jnp.float32)
  zeros_dp = jnp.zeros((STAGE_ROWS, D_OUT_PAD // NC), jnp.float32)
  zeros_c = jnp.zeros((STAGE_ROWS, CW), jnp.float32)
  ones_c = jnp.ones((CHUNK, CW), jnp.float32)

  b1_2d = b1.reshape(1, D_HID)
  grid = (N_NODES // ROW_BLK,)
  xr = pl.pallas_call(
      _tc0_body,
      grid=grid,
      in_specs=[
          pl.BlockSpec((ROW_BLK, D_IN), lambda i: (i, 0)),
          pl.BlockSpec((D_IN, D_HID), lambda i: (0, 0)),
          pl.BlockSpec((1, D_HID), lambda i: (0, 0)),
      ],
      out_specs=pl.BlockSpec((ROW_BLK, D_HID), lambda i: (i, 0)),
      out_shape=jax.ShapeDtypeStruct((N_NODES, D_HID), jnp.float32),
  )(x, W1r, b1_2d)

  s1, cnt = _sc_agg_l1(x2, edges, zeros_d1, ones_c, zeros_c)

  w2l_pad = jnp.pad(W2l, ((0, 0), (0, D_OUT_PAD - D_OUT)))
  w2r_pad = jnp.pad(W2r, ((0, 0), (0, D_OUT_PAD - D_OUT)))
  b2_pad = jnp.pad(b2, (0, D_OUT_PAD - D_OUT)).reshape(1, D_OUT_PAD)
  dh2 = D_OUT_PAD // NC
  hl2, hr, inv = pl.pallas_call(
      _tc1_body,
      grid=grid,
      in_specs=[
          pl.BlockSpec((NC, ROW_BLK, dh1), lambda i: (0, i, 0)),
          pl.BlockSpec((NC, ROW_BLK, CW), lambda i: (0, i, 0)),
          pl.BlockSpec((ROW_BLK, D_HID), lambda i: (i, 0)),
          pl.BlockSpec((D_IN, D_HID), lambda i: (0, 0)),
          pl.BlockSpec((D_HID, D_OUT_PAD), lambda i: (0, 0)),
          pl.BlockSpec((D_HID, D_OUT_PAD), lambda i: (0, 0)),
          pl.BlockSpec((1, D_OUT_PAD), lambda i: (0, 0)),
      ],
      out_specs=[
          pl.BlockSpec((ROW_BLK, D_OUT_PAD), lambda i: (i, 0)),
          pl.BlockSpec((ROW_BLK, D_OUT_PAD), lambda i: (i, 0)),
          pl.BlockSpec((ROW_BLK, 1), lambda i: (i, 0)),
      ],
      out_shape=[
          jax.ShapeDtypeStruct((N_NODES, D_OUT_PAD), jnp.float32),
          jax.ShapeDtypeStruct((N_NODES, D_OUT_PAD), jnp.float32),
          jax.ShapeDtypeStruct((N_NODES, 1), jnp.float32),
      ],
  )(s1, cnt, xr, W1l, w2l_pad, w2r_pad, b2_pad)

  (s2,) = _sc_agg_l2(hl2.reshape(NC * N_NODES, dh2), edges, zeros_dp)

  out = pl.pallas_call(
      _tc2_body,
      grid=grid,
      in_specs=[
          pl.BlockSpec((NC, ROW_BLK, dh2), lambda i: (0, i, 0)),
          pl.BlockSpec((ROW_BLK, 1), lambda i: (i, 0)),
          pl.BlockSpec((ROW_BLK, D_OUT_PAD), lambda i: (i, 0)),
      ],
      out_specs=pl.BlockSpec((ROW_BLK, D_OUT), lambda i: (i, 0)),
      out_shape=jax.ShapeDtypeStruct((N_NODES, D_OUT), jnp.float32),
  )(s2, inv, hr)

  return out


# speedup vs baseline: 14.3109x; 1.0044x over previous
import jax
import jax.numpy as jnp
from jax import lax
from jax.experimental import pallas as pl
from jax.experimental.pallas import tpu as pltpu
from jax.experimental.pallas import tpu_sc as plsc

N_NODES = 10000
N_PAD = 10240
N_EDGES = 320000
D_IN = 128
D_HID = 128
D_OUT = 40
D_OUT_PAD = 64

NC = 2
NS = 16
EDGES_PER_TILE = N_EDGES // NS
CHUNK = 80
NCHUNKS = EDGES_PER_TILE // CHUNK
NBUF = 5
ROUNDS = NCHUNKS // NBUF
CNT_PER_CORE = NCHUNKS // NC
ROWS_PER_TILE = N_PAD // NS
STAGE_ROWS = 128
CW = 8


def _make_sc_agg(d, with_cnt):
  dh = d // NC
  mesh = plsc.VectorSubcoreMesh(
      core_axis_name="c", subcore_axis_name="s", num_cores=NC,
      num_subcores=NS)

  out_type = [jax.ShapeDtypeStruct((NC, N_PAD, dh), jnp.float32)]
  scratch = [
      pltpu.VMEM((NCHUNKS, CHUNK), jnp.int32),
      pltpu.VMEM((NCHUNKS, CHUNK), jnp.int32),
      *[pltpu.VMEM((CHUNK, dh), jnp.float32) for _ in range(NBUF)],
      pltpu.VMEM((STAGE_ROWS, dh), jnp.float32),
      pltpu.VMEM_SHARED((N_PAD, dh), jnp.float32),
      *[pltpu.SemaphoreType.DMA for _ in range(2 * NBUF)],
  ]
  if with_cnt:
    out_type.append(jax.ShapeDtypeStruct((NC, N_PAD, CW), jnp.float32))
    scratch += [
        pltpu.VMEM((CHUNK, CW), jnp.float32),
        pltpu.VMEM((STAGE_ROWS, CW), jnp.float32),
        pltpu.VMEM_SHARED((N_PAD, CW), jnp.float32),
        pltpu.SemaphoreType.DMA,
    ]

  def body(*refs):
    if with_cnt:
      (table_hbm, edges_hbm, zeros_hbm, ones_hbm, zcnt_hbm,
       out_hbm, cnt_hbm,
       srcv, dstv, *rest) = refs
      rows = rest[:NBUF]
      stage, acc_sh = rest[NBUF], rest[NBUF + 1]
      gsem = rest[NBUF + 2:2 * NBUF + 2]
      ssem = rest[2 * NBUF + 2:3 * NBUF + 2]
      onesv, cstage, cacc_sh, csem = rest[3 * NBUF + 2:]
    else:
      (table_hbm, edges_hbm, zeros_hbm,
       out_hbm,
       srcv, dstv, *rest) = refs
      rows = rest[:NBUF]
      stage, acc_sh = rest[NBUF], rest[NBUF + 1]
      gsem = rest[NBUF + 2:2 * NBUF + 2]
      ssem = rest[2 * NBUF + 2:3 * NBUF + 2]

    c = lax.axis_index("c")
    s = lax.axis_index("s")

    pltpu.sync_copy(zeros_hbm, stage)
    for k in range(ROWS_PER_TILE // STAGE_ROWS):
      base = s * ROWS_PER_TILE + k * STAGE_ROWS
      pltpu.sync_copy(stage, acc_sh.at[pl.ds(base, STAGE_ROWS)])
    if with_cnt:
      pltpu.sync_copy(ones_hbm, onesv)
      pltpu.sync_copy(zcnt_hbm, cstage)
      for k in range(ROWS_PER_TILE // STAGE_ROWS):
        base = s * ROWS_PER_TILE + k * STAGE_ROWS
        pltpu.sync_copy(cstage, cacc_sh.at[pl.ds(base, STAGE_ROWS)])

    pltpu.sync_copy(edges_hbm.at[0, s], srcv)
    pltpu.sync_copy(edges_hbm.at[1, s], dstv)

    def xform(j, carry):
      for k in range(CHUNK // 16):
        v = srcv[j, pl.ds(16 * k, 16)]
        srcv[j, pl.ds(16 * k, 16)] = v + v + c
      return carry

    lax.fori_loop(0, NCHUNKS, xform, 0)
    plsc.subcore_barrier()

    for k in range(NBUF):
      pltpu.async_copy(table_hbm.at[srcv.at[k]], rows[k], gsem[k])

    def step(t, carry):
      j0 = NBUF * t
      for k in range(NBUF):
        pltpu.make_async_copy(
            table_hbm.at[srcv.at[0]], rows[k], gsem[k]).wait()
        pltpu.async_copy(rows[k], acc_sh.at[dstv.at[j0 + k]], ssem[k],
                         add=True)
      if with_cnt:
        for i in range(2):
          jc = c * CNT_PER_CORE + 2 * t + i
          pltpu.async_copy(onesv, cacc_sh.at[dstv.at[jc]], csem, add=True)
      for k in range(NBUF):
        jn = jnp.minimum(j0 + NBUF + k, NCHUNKS - 1)
        pltpu.make_async_copy(
            rows[k], acc_sh.at[dstv.at[0]], ssem[k]).wait()
        pltpu.async_copy(table_hbm.at[srcv.at[jn]], rows[k], gsem[k])
      return carry

    lax.fori_loop(0, ROUNDS, step, 0)
    if with_cnt:
      def cnt_fire(t, carry):
        jc = c * CNT_PER_CORE + 2 * ROUNDS + t
        pltpu.async_copy(onesv, cacc_sh.at[dstv.at[jc]], csem, add=True)
        return carry
      lax.fori_loop(0, CNT_PER_CORE - 2 * ROUNDS, cnt_fire, 0)
    for k in range(NBUF):
      pltpu.make_async_copy(table_hbm.at[srcv.at[0]], rows[k], gsem[k]).wait()
    if with_cnt:
      def drain(t, carry):
        pltpu.make_async_copy(onesv, cacc_sh.at[dstv.at[0]], csem).wait()
        return carry
      lax.fori_loop(0, CNT_PER_CORE, drain, 0)
    plsc.subcore_barrier()

    for k in range(ROWS_PER_TILE // STAGE_ROWS):
      base = s * ROWS_PER_TILE + k * STAGE_ROWS
      pltpu.sync_copy(acc_sh.at[pl.ds(base, STAGE_ROWS)], stage)
      pltpu.sync_copy(stage, out_hbm.at[c, pl.ds(base, STAGE_ROWS)])
      if with_cnt:
        pltpu.sync_copy(cacc_sh.at[pl.ds(base, STAGE_ROWS)], cstage)
        pltpu.sync_copy(cstage, cnt_hbm.at[c, pl.ds(base, STAGE_ROWS)])

  return pl.kernel(body, out_type=out_type, mesh=mesh,
                   scratch_types=scratch,
                   compiler_params=pltpu.CompilerParams(
                       use_tc_tiling_on_sc=False))


_sc_agg_l1 = _make_sc_agg(D_IN, with_cnt=True)
_sc_agg_l2 = _make_sc_agg(D_OUT_PAD, with_cnt=False)


ROW_BLK = 2000


def _tc0_body(x_ref, w1r_ref, b1_ref, xr_ref):
  xr_ref[...] = x_ref[...] @ w1r_ref[...] + b1_ref[...]


def _tc1_body(s1_ref, cnt_ref, xr_ref, w1l_ref, w2l_ref,
              w2r_ref, b2_ref, hl_ref, hr_ref, inv_ref):
  cnt = cnt_ref[0, :, 0:1] + cnt_ref[1, :, 0:1]
  inv = 1.0 / jnp.maximum(cnt, 1.0)
  inv_ref[...] = inv
  summ = jnp.concatenate([s1_ref[0], s1_ref[1]], axis=1)
  mean = summ * inv
  h = jnp.maximum(mean @ w1l_ref[...] + xr_ref[...], 0.0)
  hl_ref[...] = h @ w2l_ref[...]
  hr_ref[...] = h @ w2r_ref[...] + b2_ref[...]


def _tc2_body(s2_ref, inv_ref, hr_ref, out_ref):
  s2 = jnp.concatenate([s2_ref[0], s2_ref[1]], axis=1)
  out_ref[...] = (s2 * inv_ref[...] + hr_ref[...])[:, :D_OUT]


@jax.jit
def kernel(x, edge_index, W1l, W1r, b1, W2l, W2r, b2):
  edges = edge_index.reshape(2, NS, NCHUNKS, CHUNK)
  dh1 = D_IN // NC
  x2 = x.reshape(NC * N_NODES, dh1)
  zeros_d1 = jnp.zeros((STAGE_ROWS, dh1), jnp.float32)
  zeros_dp = jnp.zeros((STAGE_ROWS, D_OUT_PAD // NC), jnp.float32)
  zeros_c = jnp.zeros((STAGE_ROWS, CW), jnp.float32)
  ones_c = jnp.ones((CHUNK, CW), jnp.float32)

  b1_2d = b1.reshape(1, D_HID)
  grid = (N_NODES // ROW_BLK,)
  xr = pl.pallas_call(
      _tc0_body,
      grid=grid,
      in_specs=[
          pl.BlockSpec((ROW_BLK, D_IN), lambda i: (i, 0)),
          pl.BlockSpec((D_IN, D_HID), lambda i: (0, 0)),
          pl.BlockSpec((1, D_HID), lambda i: (0, 0)),
      ],
      out_specs=pl.BlockSpec((ROW_BLK, D_HID), lambda i: (i, 0)),
      out_shape=jax.ShapeDtypeStruct((N_NODES, D_HID), jnp.float32),
  )(x, W1r, b1_2d)

  s1, cnt = _sc_agg_l1(x2, edges, zeros_d1, ones_c, zeros_c)

  w2l_pad = jnp.pad(W2l, ((0, 0), (0, D_OUT_PAD - D_OUT)))
  w2r_pad = jnp.pad(W2r, ((0, 0), (0, D_OUT_PAD - D_OUT)))
  b2_pad = jnp.pad(b2, (0, D_OUT_PAD - D_OUT)).reshape(1, D_OUT_PAD)
  dh2 = D_OUT_PAD // NC
  hl2, hr, inv = pl.pallas_call(
      _tc1_body,
      grid=grid,
      in_specs=[
          pl.BlockSpec((NC, ROW_BLK, dh1), lambda i: (0, i, 0)),
          pl.BlockSpec((NC, ROW_BLK, CW), lambda i: (0, i, 0)),
          pl.BlockSpec((ROW_BLK, D_HID), lambda i: (i, 0)),
          pl.BlockSpec((D_IN, D_HID), lambda i: (0, 0)),
          pl.BlockSpec((D_HID, D_OUT_PAD), lambda i: (0, 0)),
          pl.BlockSpec((D_HID, D_OUT_PAD), lambda i: (0, 0)),
          pl.BlockSpec((1, D_OUT_PAD), lambda i: (0, 0)),
      ],
      out_specs=[
          pl.BlockSpec((ROW_BLK, D_OUT_PAD), lambda i: (i, 0)),
          pl.BlockSpec((ROW_BLK, D_OUT_PAD), lambda i: (i, 0)),
          pl.BlockSpec((ROW_BLK, 1), lambda i: (i, 0)),
      ],
      out_shape=[
          jax.ShapeDtypeStruct((N_NODES, D_OUT_PAD), jnp.float32),
          jax.ShapeDtypeStruct((N_NODES, D_OUT_PAD), jnp.float32),
          jax.ShapeDtypeStruct((N_NODES, 1), jnp.float32),
      ],
  )(s1, cnt, xr, W1l, w2l_pad, w2r_pad, b2_pad)

  (s2,) = _sc_agg_l2(hl2.reshape(NC * N_NODES, dh2), edges, zeros_dp)

  out = pl.pallas_call(
      _tc2_body,
      grid=grid,
      in_specs=[
          pl.BlockSpec((NC, ROW_BLK, dh2), lambda i: (0, i, 0)),
          pl.BlockSpec((ROW_BLK, 1), lambda i: (i, 0)),
          pl.BlockSpec((ROW_BLK, D_OUT_PAD), lambda i: (i, 0)),
      ],
      out_specs=pl.BlockSpec((ROW_BLK, D_OUT), lambda i: (i, 0)),
      out_shape=jax.ShapeDtypeStruct((N_NODES, D_OUT), jnp.float32),
  )(s2, inv, hr)

  return out


# speedup vs baseline: 14.3338x; 1.0016x over previous
import jax
import jax.numpy as jnp
from jax import lax
from jax.experimental import pallas as pl
from jax.experimental.pallas import tpu as pltpu
from jax.experimental.pallas import tpu_sc as plsc

N_NODES = 10000
N_PAD = 10240
N_EDGES = 320000
D_IN = 128
D_HID = 128
D_OUT = 40
D_OUT_PAD = 64

NC = 2
NS = 16
EDGES_PER_TILE = N_EDGES // NS
CHUNK = 80
NCHUNKS = EDGES_PER_TILE // CHUNK
NBUF = 5
ROUNDS = NCHUNKS // NBUF
CNT_PER_CORE = NCHUNKS // NC
ROWS_PER_TILE = N_PAD // NS
STAGE_ROWS = 128
CW = 8


def _make_sc_agg(d, with_cnt):
  dh = d // NC
  mesh = plsc.VectorSubcoreMesh(
      core_axis_name="c", subcore_axis_name="s", num_cores=NC,
      num_subcores=NS)

  dout = dh + CW if with_cnt else dh
  out_type = [jax.ShapeDtypeStruct((NC, N_PAD, dout), jnp.float32)]
  scratch = [
      pltpu.VMEM((NCHUNKS, CHUNK), jnp.int32),
      pltpu.VMEM((NCHUNKS, CHUNK), jnp.int32),
      *[pltpu.VMEM((CHUNK, dh), jnp.float32) for _ in range(NBUF)],
      pltpu.VMEM((STAGE_ROWS, dh), jnp.float32),
      pltpu.VMEM_SHARED((N_PAD, dh), jnp.float32),
      *[pltpu.SemaphoreType.DMA for _ in range(2 * NBUF)],
  ]
  if with_cnt:
    scratch += [
        pltpu.VMEM((CHUNK, CW), jnp.float32),
        pltpu.VMEM((STAGE_ROWS, CW), jnp.float32),
        pltpu.VMEM_SHARED((N_PAD, CW), jnp.float32),
        pltpu.SemaphoreType.DMA,
    ]

  def body(*refs):
    if with_cnt:
      (table_hbm, edges_hbm, zeros_hbm, ones_hbm, zcnt_hbm,
       out_hbm,
       srcv, dstv, *rest) = refs
      rows = rest[:NBUF]
      stage, acc_sh = rest[NBUF], rest[NBUF + 1]
      gsem = rest[NBUF + 2:2 * NBUF + 2]
      ssem = rest[2 * NBUF + 2:3 * NBUF + 2]
      onesv, cstage, cacc_sh, csem = rest[3 * NBUF + 2:]
    else:
      (table_hbm, edges_hbm, zeros_hbm,
       out_hbm,
       srcv, dstv, *rest) = refs
      rows = rest[:NBUF]
      stage, acc_sh = rest[NBUF], rest[NBUF + 1]
      gsem = rest[NBUF + 2:2 * NBUF + 2]
      ssem = rest[2 * NBUF + 2:3 * NBUF + 2]

    c = lax.axis_index("c")
    s = lax.axis_index("s")

    pltpu.sync_copy(zeros_hbm, stage)
    for k in range(ROWS_PER_TILE // STAGE_ROWS):
      base = s * ROWS_PER_TILE + k * STAGE_ROWS
      pltpu.sync_copy(stage, acc_sh.at[pl.ds(base, STAGE_ROWS)])
    if with_cnt:
      pltpu.sync_copy(ones_hbm, onesv)
      pltpu.sync_copy(zcnt_hbm, cstage)
      for k in range(ROWS_PER_TILE // STAGE_ROWS):
        base = s * ROWS_PER_TILE + k * STAGE_ROWS
        pltpu.sync_copy(cstage, cacc_sh.at[pl.ds(base, STAGE_ROWS)])

    pltpu.sync_copy(edges_hbm.at[0, s], srcv)
    pltpu.sync_copy(edges_hbm.at[1, s], dstv)

    def xform(j, carry):
      for k in range(CHUNK // 16):
        v = srcv[j, pl.ds(16 * k, 16)]
        srcv[j, pl.ds(16 * k, 16)] = v + v + c
      return carry

    lax.fori_loop(0, NCHUNKS, xform, 0)
    plsc.subcore_barrier()

    for k in range(NBUF):
      pltpu.async_copy(table_hbm.at[srcv.at[k]], rows[k], gsem[k])

    def step(t, carry):
      j0 = NBUF * t
      for k in range(NBUF):
        pltpu.make_async_copy(
            table_hbm.at[srcv.at[0]], rows[k], gsem[k]).wait()
        pltpu.async_copy(rows[k], acc_sh.at[dstv.at[j0 + k]], ssem[k],
                         add=True)
      if with_cnt:
        for i in range(2):
          jc = c * CNT_PER_CORE + 2 * t + i
          pltpu.async_copy(onesv, cacc_sh.at[dstv.at[jc]], csem, add=True)
      for k in range(NBUF):
        jn = jnp.minimum(j0 + NBUF + k, NCHUNKS - 1)
        pltpu.make_async_copy(
            rows[k], acc_sh.at[dstv.at[0]], ssem[k]).wait()
        pltpu.async_copy(table_hbm.at[srcv.at[jn]], rows[k], gsem[k])
      return carry

    lax.fori_loop(0, ROUNDS, step, 0)
    if with_cnt:
      def cnt_fire(t, carry):
        jc = c * CNT_PER_CORE + 2 * ROUNDS + t
        pltpu.async_copy(onesv, cacc_sh.at[dstv.at[jc]], csem, add=True)
        return carry
      lax.fori_loop(0, CNT_PER_CORE - 2 * ROUNDS, cnt_fire, 0)
    for k in range(NBUF):
      pltpu.make_async_copy(table_hbm.at[srcv.at[0]], rows[k], gsem[k]).wait()
    if with_cnt:
      def drain(t, carry):
        pltpu.make_async_copy(onesv, cacc_sh.at[dstv.at[0]], csem).wait()
        return carry
      lax.fori_loop(0, CNT_PER_CORE, drain, 0)
    plsc.subcore_barrier()

    for k in range(ROWS_PER_TILE // STAGE_ROWS):
      base = s * ROWS_PER_TILE + k * STAGE_ROWS
      pltpu.sync_copy(acc_sh.at[pl.ds(base, STAGE_ROWS)], stage)
      if with_cnt:
        pltpu.sync_copy(
            stage, out_hbm.at[c, pl.ds(base, STAGE_ROWS), pl.ds(0, dh)])
        pltpu.sync_copy(cacc_sh.at[pl.ds(base, STAGE_ROWS)], cstage)
        pltpu.sync_copy(
            cstage, out_hbm.at[c, pl.ds(base, STAGE_ROWS), pl.ds(dh, CW)])
      else:
        pltpu.sync_copy(stage, out_hbm.at[c, pl.ds(base, STAGE_ROWS)])

  return pl.kernel(body, out_type=out_type, mesh=mesh,
                   scratch_types=scratch,
                   compiler_params=pltpu.CompilerParams(
                       use_tc_tiling_on_sc=False))


_sc_agg_l1 = _make_sc_agg(D_IN, with_cnt=True)
_sc_agg_l2 = _make_sc_agg(D_OUT_PAD, with_cnt=False)


ROW_BLK = 2000


def _tc0_body(x_ref, w1r_ref, b1_ref, xr_ref):
  xr_ref[...] = x_ref[...] @ w1r_ref[...] + b1_ref[...]


def _tc1_body(s1_ref, xr_ref, w1l_ref, w2l_ref,
              w2r_ref, b2_ref, hl_ref, hr_ref, inv_ref):
  dh1 = D_IN // NC
  cnt = s1_ref[0, :, dh1:dh1 + 1] + s1_ref[1, :, dh1:dh1 + 1]
  inv = 1.0 / jnp.maximum(cnt, 1.0)
  inv_ref[...] = inv
  summ = jnp.concatenate(
      [s1_ref[0, :, :dh1], s1_ref[1, :, :dh1]], axis=1)
  mean = summ * inv
  h = jnp.maximum(mean @ w1l_ref[...] + xr_ref[...], 0.0)
  hl_ref[...] = h @ w2l_ref[...]
  hr_ref[...] = h @ w2r_ref[...] + b2_ref[...]


def _tc2_body(s2_ref, inv_ref, hr_ref, out_ref):
  s2 = jnp.concatenate([s2_ref[0], s2_ref[1]], axis=1)
  out_ref[...] = (s2 * inv_ref[...] + hr_ref[...])[:, :D_OUT]


@jax.jit
def kernel(x, edge_index, W1l, W1r, b1, W2l, W2r, b2):
  edges = edge_index.reshape(2, NS, NCHUNKS, CHUNK)
  dh1 = D_IN // NC
  x2 = x.reshape(NC * N_NODES, dh1)
  zeros_d1 = jnp.zeros((STAGE_ROWS, dh1), jnp.float32)
  zeros_dp = jnp.zeros((STAGE_ROWS, D_OUT_PAD // NC), jnp.float32)
  zeros_c = jnp.zeros((STAGE_ROWS, CW), jnp.float32)
  ones_c = jnp.ones((CHUNK, CW), jnp.float32)

  b1_2d = b1.reshape(1, D_HID)
  grid = (N_NODES // ROW_BLK,)
  xr = pl.pallas_call(
      _tc0_body,
      grid=grid,
      in_specs=[
          pl.BlockSpec((ROW_BLK, D_IN), lambda i: (i, 0)),
          pl.BlockSpec((D_IN, D_HID), lambda i: (0, 0)),
          pl.BlockSpec((1, D_HID), lambda i: (0, 0)),
      ],
      out_specs=pl.BlockSpec((ROW_BLK, D_HID), lambda i: (i, 0)),
      out_shape=jax.ShapeDtypeStruct((N_NODES, D_HID), jnp.float32),
  )(x, W1r, b1_2d)

  (s1,) = _sc_agg_l1(x2, edges, zeros_d1, ones_c, zeros_c)

  w2l_pad = jnp.pad(W2l, ((0, 0), (0, D_OUT_PAD - D_OUT)))
  w2r_pad = jnp.pad(W2r, ((0, 0), (0, D_OUT_PAD - D_OUT)))
  b2_pad = jnp.pad(b2, (0, D_OUT_PAD - D_OUT)).reshape(1, D_OUT_PAD)
  dh2 = D_OUT_PAD // NC
  hl2, hr, inv = pl.pallas_call(
      _tc1_body,
      grid=grid,
      in_specs=[
          pl.BlockSpec((NC, ROW_BLK, dh1 + CW), lambda i: (0, i, 0)),
          pl.BlockSpec((ROW_BLK, D_HID), lambda i: (i, 0)),
          pl.BlockSpec((D_IN, D_HID), lambda i: (0, 0)),
          pl.BlockSpec((D_HID, D_OUT_PAD), lambda i: (0, 0)),
          pl.BlockSpec((D_HID, D_OUT_PAD), lambda i: (0, 0)),
          pl.BlockSpec((1, D_OUT_PAD), lambda i: (0, 0)),
      ],
      out_specs=[
          pl.BlockSpec((ROW_BLK, D_OUT_PAD), lambda i: (i, 0)),
          pl.BlockSpec((ROW_BLK, D_OUT_PAD), lambda i: (i, 0)),
          pl.BlockSpec((ROW_BLK, 1), lambda i: (i, 0)),
      ],
      out_shape=[
          jax.ShapeDtypeStruct((N_NODES, D_OUT_PAD), jnp.float32),
          jax.ShapeDtypeStruct((N_NODES, D_OUT_PAD), jnp.float32),
          jax.ShapeDtypeStruct((N_NODES, 1), jnp.float32),
      ],
  )(s1, xr, W1l, w2l_pad, w2r_pad, b2_pad)

  (s2,) = _sc_agg_l2(hl2.reshape(NC * N_NODES, dh2), edges, zeros_dp)

  out = pl.pallas_call(
      _tc2_body,
      grid=grid,
      in_specs=[
          pl.BlockSpec((NC, ROW_BLK, dh2), lambda i: (0, i, 0)),
          pl.BlockSpec((ROW_BLK, 1), lambda i: (i, 0)),
          pl.BlockSpec((ROW_BLK, D_OUT_PAD), lambda i: (i, 0)),
      ],
      out_specs=pl.BlockSpec((ROW_BLK, D_OUT), lambda i: (i, 0)),
      out_shape=jax.ShapeDtypeStruct((N_NODES, D_OUT), jnp.float32),
  )(s2, inv, hr)

  return out


# speedup vs baseline: 14.4477x; 1.0079x over previous
import jax
import jax.numpy as jnp
from jax import lax
from jax.experimental import pallas as pl
from jax.experimental.pallas import tpu as pltpu
from jax.experimental.pallas import tpu_sc as plsc

N_NODES = 10000
N_PAD = 10240
N_EDGES = 320000
D_IN = 128
D_HID = 128
D_OUT = 40
D_OUT_PAD = 48

NC = 2
NS = 16
EDGES_PER_TILE = N_EDGES // NS
CHUNK = 80
NCHUNKS = EDGES_PER_TILE // CHUNK
NBUF = 5
ROUNDS = NCHUNKS // NBUF
CNT_PER_CORE = NCHUNKS // NC
ROWS_PER_TILE = N_PAD // NS
STAGE_ROWS = 128
CW = 8


def _make_sc_agg(d, with_cnt):
  dh = d // NC
  mesh = plsc.VectorSubcoreMesh(
      core_axis_name="c", subcore_axis_name="s", num_cores=NC,
      num_subcores=NS)

  out_type = [jax.ShapeDtypeStruct((NC, N_PAD, dh), jnp.float32)]
  scratch = [
      pltpu.VMEM((NCHUNKS, CHUNK), jnp.int32),
      pltpu.VMEM((NCHUNKS, CHUNK), jnp.int32),
      *[pltpu.VMEM((CHUNK, dh), jnp.float32) for _ in range(NBUF)],
      pltpu.VMEM((STAGE_ROWS, dh), jnp.float32),
      pltpu.VMEM_SHARED((N_PAD, dh), jnp.float32),
      *[pltpu.SemaphoreType.DMA for _ in range(2 * NBUF)],
  ]
  if with_cnt:
    out_type.append(jax.ShapeDtypeStruct((NC, N_PAD, CW), jnp.float32))
    scratch += [
        pltpu.VMEM((CHUNK, CW), jnp.float32),
        pltpu.VMEM((STAGE_ROWS, CW), jnp.float32),
        pltpu.VMEM_SHARED((N_PAD, CW), jnp.float32),
        pltpu.SemaphoreType.DMA,
    ]

  def body(*refs):
    if with_cnt:
      (table_hbm, edges_hbm, zeros_hbm, ones_hbm, zcnt_hbm,
       out_hbm, cnt_hbm,
       srcv, dstv, *rest) = refs
      rows = rest[:NBUF]
      stage, acc_sh = rest[NBUF], rest[NBUF + 1]
      gsem = rest[NBUF + 2:2 * NBUF + 2]
      ssem = rest[2 * NBUF + 2:3 * NBUF + 2]
      onesv, cstage, cacc_sh, csem = rest[3 * NBUF + 2:]
    else:
      (table_hbm, edges_hbm, zeros_hbm,
       out_hbm,
       srcv, dstv, *rest) = refs
      rows = rest[:NBUF]
      stage, acc_sh = rest[NBUF], rest[NBUF + 1]
      gsem = rest[NBUF + 2:2 * NBUF + 2]
      ssem = rest[2 * NBUF + 2:3 * NBUF + 2]

    c = lax.axis_index("c")
    s = lax.axis_index("s")

    pltpu.sync_copy(zeros_hbm, stage)
    for k in range(ROWS_PER_TILE // STAGE_ROWS):
      base = s * ROWS_PER_TILE + k * STAGE_ROWS
      pltpu.sync_copy(stage, acc_sh.at[pl.ds(base, STAGE_ROWS)])
    if with_cnt:
      pltpu.sync_copy(ones_hbm, onesv)
      pltpu.sync_copy(zcnt_hbm, cstage)
      for k in range(ROWS_PER_TILE // STAGE_ROWS):
        base = s * ROWS_PER_TILE + k * STAGE_ROWS
        pltpu.sync_copy(cstage, cacc_sh.at[pl.ds(base, STAGE_ROWS)])

    pltpu.sync_copy(edges_hbm.at[0, s], srcv)
    pltpu.sync_copy(edges_hbm.at[1, s], dstv)

    def xform(j, carry):
      for k in range(CHUNK // 16):
        v = srcv[j, pl.ds(16 * k, 16)]
        srcv[j, pl.ds(16 * k, 16)] = v + v + c
      return carry

    lax.fori_loop(0, NCHUNKS, xform, 0)
    plsc.subcore_barrier()

    for k in range(NBUF):
      pltpu.async_copy(table_hbm.at[srcv.at[k]], rows[k], gsem[k])

    def step(t, carry):
      j0 = NBUF * t
      for k in range(NBUF):
        pltpu.make_async_copy(
            table_hbm.at[srcv.at[0]], rows[k], gsem[k]).wait()
        pltpu.async_copy(rows[k], acc_sh.at[dstv.at[j0 + k]], ssem[k],
                         add=True)
      if with_cnt:
        for i in range(2):
          jc = c * CNT_PER_CORE + 2 * t + i
          pltpu.async_copy(onesv, cacc_sh.at[dstv.at[jc]], csem, add=True)
      for k in range(NBUF):
        jn = jnp.minimum(j0 + NBUF + k, NCHUNKS - 1)
        pltpu.make_async_copy(
            rows[k], acc_sh.at[dstv.at[0]], ssem[k]).wait()
        pltpu.async_copy(table_hbm.at[srcv.at[jn]], rows[k], gsem[k])
      return carry

    lax.fori_loop(0, ROUNDS, step, 0)
    if with_cnt:
      def cnt_fire(t, carry):
        jc = c * CNT_PER_CORE + 2 * ROUNDS + t
        pltpu.async_copy(onesv, cacc_sh.at[dstv.at[jc]], csem, add=True)
        return carry
      lax.fori_loop(0, CNT_PER_CORE - 2 * ROUNDS, cnt_fire, 0)
    for k in range(NBUF):
      pltpu.make_async_copy(table_hbm.at[srcv.at[0]], rows[k], gsem[k]).wait()
    if with_cnt:
      def drain(t, carry):
        pltpu.make_async_copy(onesv, cacc_sh.at[dstv.at[0]], csem).wait()
        return carry
      lax.fori_loop(0, CNT_PER_CORE, drain, 0)
    plsc.subcore_barrier()

    for k in range(ROWS_PER_TILE // STAGE_ROWS):
      base = s * ROWS_PER_TILE + k * STAGE_ROWS
      pltpu.sync_copy(acc_sh.at[pl.ds(base, STAGE_ROWS)], stage)
      pltpu.sync_copy(stage, out_hbm.at[c, pl.ds(base, STAGE_ROWS)])
      if with_cnt:
        pltpu.sync_copy(cacc_sh.at[pl.ds(base, STAGE_ROWS)], cstage)
        pltpu.sync_copy(cstage, cnt_hbm.at[c, pl.ds(base, STAGE_ROWS)])

  return pl.kernel(body, out_type=out_type, mesh=mesh,
                   scratch_types=scratch,
                   compiler_params=pltpu.CompilerParams(
                       use_tc_tiling_on_sc=False))


_sc_agg_l1 = _make_sc_agg(D_IN, with_cnt=True)
_sc_agg_l2 = _make_sc_agg(D_OUT_PAD, with_cnt=False)


ROW_BLK = 2000


def _tc0_body(x_ref, w1r_ref, b1_ref, xr_ref):
  xr_ref[...] = x_ref[...] @ w1r_ref[...] + b1_ref[...]


def _tc1_body(s1_ref, cnt_ref, xr_ref, w1l_ref, w2l_ref,
              w2r_ref, b2_ref, hl_ref, hr_ref, inv_ref):
  cnt = cnt_ref[0, :, 0:1] + cnt_ref[1, :, 0:1]
  inv = 1.0 / jnp.maximum(cnt, 1.0)
  inv_ref[...] = inv
  summ = jnp.concatenate([s1_ref[0], s1_ref[1]], axis=1)
  mean = summ * inv
  h = jnp.maximum(mean @ w1l_ref[...] + xr_ref[...], 0.0)
  hl_ref[...] = h @ w2l_ref[...]
  hr_ref[...] = h @ w2r_ref[...] + b2_ref[...]


def _tc2_body(s2_ref, inv_ref, hr_ref, out_ref):
  s2 = jnp.concatenate([s2_ref[0], s2_ref[1]], axis=1)
  out_ref[...] = (s2 * inv_ref[...] + hr_ref[...])[:, :D_OUT]


@jax.jit
def kernel(x, edge_index, W1l, W1r, b1, W2l, W2r, b2):
  edges = edge_index.reshape(2, NS, NCHUNKS, CHUNK)
  dh1 = D_IN // NC
  x2 = x.reshape(NC * N_NODES, dh1)
  zeros_d1 = jnp.zeros((STAGE_ROWS, dh1), jnp.float32)
  zeros_dp = jnp.zeros((STAGE_ROWS, D_OUT_PAD // NC), jnp.float32)
  zeros_c = jnp.zeros((STAGE_ROWS, CW), jnp.float32)
  ones_c = jnp.ones((CHUNK, CW), jnp.float32)

  b1_2d = b1.reshape(1, D_HID)
  grid = (N_NODES // ROW_BLK,)
  xr = pl.pallas_call(
      _tc0_body,
      grid=grid,
      in_specs=[
          pl.BlockSpec((ROW_BLK, D_IN), lambda i: (i, 0)),
          pl.BlockSpec((D_IN, D_HID), lambda i: (0, 0)),
          pl.BlockSpec((1, D_HID), lambda i: (0, 0)),
      ],
      out_specs=pl.BlockSpec((ROW_BLK, D_HID), lambda i: (i, 0)),
      out_shape=jax.ShapeDtypeStruct((N_NODES, D_HID), jnp.float32),
  )(x, W1r, b1_2d)

  s1, cnt = _sc_agg_l1(x2, edges, zeros_d1, ones_c, zeros_c)

  w2l_pad = jnp.pad(W2l, ((0, 0), (0, D_OUT_PAD - D_OUT)))
  w2r_pad = jnp.pad(W2r, ((0, 0), (0, D_OUT_PAD - D_OUT)))
  b2_pad = jnp.pad(b2, (0, D_OUT_PAD - D_OUT)).reshape(1, D_OUT_PAD)
  dh2 = D_OUT_PAD // NC
  hl2, hr, inv = pl.pallas_call(
      _tc1_body,
      grid=grid,
      in_specs=[
          pl.BlockSpec((NC, ROW_BLK, dh1), lambda i: (0, i, 0)),
          pl.BlockSpec((NC, ROW_BLK, CW), lambda i: (0, i, 0)),
          pl.BlockSpec((ROW_BLK, D_HID), lambda i: (i, 0)),
          pl.BlockSpec((D_IN, D_HID), lambda i: (0, 0)),
          pl.BlockSpec((D_HID, D_OUT_PAD), lambda i: (0, 0)),
          pl.BlockSpec((D_HID, D_OUT_PAD), lambda i: (0, 0)),
          pl.BlockSpec((1, D_OUT_PAD), lambda i: (0, 0)),
      ],
      out_specs=[
          pl.BlockSpec((ROW_BLK, D_OUT_PAD), lambda i: (i, 0)),
          pl.BlockSpec((ROW_BLK, D_OUT_PAD), lambda i: (i, 0)),
          pl.BlockSpec((ROW_BLK, 1), lambda i: (i, 0)),
      ],
      out_shape=[
          jax.ShapeDtypeStruct((N_NODES, D_OUT_PAD), jnp.float32),
          jax.ShapeDtypeStruct((N_NODES, D_OUT_PAD), jnp.float32),
          jax.ShapeDtypeStruct((N_NODES, 1), jnp.float32),
      ],
  )(s1, cnt, xr, W1l, w2l_pad, w2r_pad, b2_pad)

  (s2,) = _sc_agg_l2(hl2.reshape(NC * N_NODES, dh2), edges, zeros_dp)

  out = pl.pallas_call(
      _tc2_body,
      grid=grid,
      in_specs=[
          pl.BlockSpec((NC, ROW_BLK, dh2), lambda i: (0, i, 0)),
          pl.BlockSpec((ROW_BLK, 1), lambda i: (i, 0)),
          pl.BlockSpec((ROW_BLK, D_OUT_PAD), lambda i: (i, 0)),
      ],
      out_specs=pl.BlockSpec((ROW_BLK, D_OUT), lambda i: (i, 0)),
      out_shape=jax.ShapeDtypeStruct((N_NODES, D_OUT), jnp.float32),
  )(s2, inv, hr)

  return out
